# baseline (speedup 1.0000x reference)
import functools

import jax
import jax.numpy as jnp
from jax.experimental import pallas as pl
from jax.experimental.pallas import tpu as pltpu

_NFILT = 40
_FPAD = 64


def _frontend_body(nfilt, b0_ref, b1_ref, b2_ref, x_ref, o_ref):
    nb = x_ref.shape[-1]
    fp = o_ref.shape[-1]
    b0, b1, b2 = b0_ref[...], b1_ref[...], b2_ref[...]
    f0, f1, f2 = jnp.floor(b0), jnp.floor(b1), jnp.floor(b2)
    i = jax.lax.broadcasted_iota(jnp.int32, (nb, fp), 0).astype(jnp.float32)
    j = jax.lax.broadcasted_iota(jnp.int32, (nb, fp), 1)
    rise_m = (i >= f0) & (i < f1)
    fall_m = (i >= f1) & (i < f2)
    d1 = b1 - b0
    d2 = b2 - b1
    rv = (i - b0) / jnp.where(d1 > 0, d1, 1.0) ** 2
    fv = (b2 - i) / jnp.where(d2 > 0, d2, 1.0) ** 2
    val = jnp.where(fall_m, fv, jnp.where(rise_m, rv, 0.0))
    val = jnp.where(j < nfilt - 1, val, 0.0)
    val = jnp.where(j == 0, jnp.where(i == 0.0, 1.0, 0.0), val)

    bh, tc, _ = x_ref.shape
    xb = x_ref[...].reshape(bh * tc, nb)
    filt = jnp.dot(xb, val, preferred_element_type=jnp.float32)
    h = jnp.log(filt + 1e-10).astype(jnp.bfloat16)
    o_ref[...] = h.reshape(bh, tc, fp)


def _frontend(x, binpoints, bh, tcf):
    B, T, NB = x.shape
    nt = T // tcf
    pad = _FPAD - _NFILT
    b0 = jnp.pad(binpoints[0:_NFILT], (0, pad)).reshape(1, _FPAD)
    b1 = jnp.pad(binpoints[1:_NFILT + 1], (0, pad)).reshape(1, _FPAD)
    b2 = jnp.pad(binpoints[2:_NFILT + 2], (0, pad)).reshape(1, _FPAD)
    return pl.pallas_call(
        functools.partial(_frontend_body, _NFILT),
        grid=(B // bh, nt),
        in_specs=[
            pl.BlockSpec((1, _FPAD), lambda b, k: (0, 0)),
            pl.BlockSpec((1, _FPAD), lambda b, k: (0, 0)),
            pl.BlockSpec((1, _FPAD), lambda b, k: (0, 0)),
            pl.BlockSpec((bh, tcf, NB), lambda b, k: (b, k, 0)),
        ],
        out_specs=pl.BlockSpec((bh, tcf, _FPAD), lambda b, k: (b, k, 0)),
        out_shape=jax.ShapeDtypeStruct((B, T, _FPAD), jnp.bfloat16),
        compiler_params=pltpu.CompilerParams(
            dimension_semantics=("parallel", "arbitrary")),
    )(b0, b1, b2, x)


def _lstm_body(tc, bh, hid2, n_in, accumulate, t_total, nt, *refs):
    g8 = 4 * hid2
    xf = refs[0:n_in]
    xb = refs[n_in:2 * n_in]
    wf = refs[2 * n_in:3 * n_in]
    wb = refs[3 * n_in:4 * n_in]
    wc, bf, bb = refs[4 * n_in:4 * n_in + 3]
    n_out = 1 if accumulate else 2
    outs = refs[4 * n_in + 3:4 * n_in + 3 + n_out]
    pf_s, pb_s, h_s, c_s, a_s = refs[4 * n_in + 3 + n_out:]
    k = pl.program_id(1)

    @pl.when(k == 0)
    def _():
        h_s[...] = jnp.zeros_like(h_s)
        c_s[...] = jnp.zeros_like(c_s)
        a_s[...] = jnp.zeros_like(a_s)

    def make_pre(xs, ws, b_ref):
        acc = b_ref[...]
        for xr, w_ref in zip(xs, ws):
            d = xr.shape[-1]
            x2 = xr[...].reshape(tc * bh, d)
            acc = acc + jnp.dot(x2, w_ref[...],
                                preferred_element_type=jnp.float32)
        return acc

    pf_s[...] = make_pre(xf, wf, bf)
    pb_s[...] = make_pre(xb, wb, bb)

    wc_v = wc[...]
    ns = 2
    sb = bh // ns

    def step(t, carry):
        hs, cs, acs = carry
        tb = tc - 1 - t
        new_h, new_c, new_a, new_hf = [], [], [], []
        for s in range(ns):
            rf = pl.multiple_of(t * bh + s * sb, sb)
            rb = pl.multiple_of(tb * bh + s * sb, sb)
            g = (pf_s[pl.ds(rf, sb), :] + pb_s[pl.ds(rb, sb), :]
                 + jnp.dot(hs[s], wc_v, preferred_element_type=jnp.float32))
            ig = jax.nn.sigmoid(g[:, 0:hid2])
            fg = jax.nn.sigmoid(g[:, hid2:2 * hid2])
            gg = jnp.tanh(g[:, 2 * hid2:3 * hid2])
            og = jax.nn.sigmoid(g[:, 3 * hid2:4 * hid2])
            c = fg * cs[s] + ig * gg
            hf32 = og * jnp.tanh(c)
            new_c.append(c)
            new_hf.append(hf32)
            new_h.append(hf32.astype(jnp.bfloat16))
            if accumulate:
                new_a.append(acs[s] + hf32)
        if not accumulate:
            hcat = jnp.concatenate(new_h, axis=0)
            outs[0][pl.ds(t, 1)] = hcat.reshape(1, bh, hid2)
            outs[1][pl.ds(tb, 1)] = hcat.reshape(1, bh, hid2)
            new_a = list(acs)
        return (tuple(new_h), tuple(new_c), tuple(new_a))

    init = (tuple(h_s[pl.ds(s * sb, sb), :] for s in range(ns)),
            tuple(c_s[pl.ds(s * sb, sb), :] for s in range(ns)),
            tuple(a_s[pl.ds(s * sb, sb), :] for s in range(ns)))
    fin = jax.lax.fori_loop(0, tc, step, init)
    h_s[...] = jnp.concatenate(fin[0], axis=0)
    c_s[...] = jnp.concatenate(fin[1], axis=0)
    a_s[...] = jnp.concatenate(fin[2], axis=0)

    if accumulate:
        @pl.when(k == nt - 1)
        def _():
            outs[0][...] = jnp.concatenate(fin[2], axis=0) * (1.0 / t_total)


def _lstm_layer(ins, wf_list, wb_list, wc, bf, bb, bh, tc, accumulate):
    T, B, _ = ins[0].shape
    hid2 = wc.shape[0]
    g8 = wc.shape[1]
    nt = T // tc
    nb = B // bh
    n_in = len(ins)

    in_specs = []
    args = []
    for xr in ins:
        d = xr.shape[-1]
        in_specs.append(pl.BlockSpec((tc, bh, d), lambda b, k: (k, b, 0)))
        args.append(xr)
    for xr in ins:
        d = xr.shape[-1]
        in_specs.append(
            pl.BlockSpec((tc, bh, d), lambda b, k: (nt - 1 - k, b, 0)))
        args.append(xr)
    for w in (*wf_list, *wb_list, wc, bf, bb):
        in_specs.append(pl.BlockSpec(w.shape, lambda b, k: (0,) * w.ndim))
        args.append(w)

    if accumulate:
        out_shape = (jax.ShapeDtypeStruct((B, hid2), jnp.float32),)
        out_specs = [pl.BlockSpec((bh, hid2), lambda b, k: (b, 0))]
    else:
        out_shape = (jax.ShapeDtypeStruct((T, B, hid2), jnp.bfloat16),) * 2
        out_specs = [
            pl.BlockSpec((tc, bh, hid2), lambda b, k: (k, b, 0)),
            pl.BlockSpec((tc, bh, hid2), lambda b, k: (nt - 1 - k, b, 0)),
        ]

    scratch = [
        pltpu.VMEM((tc * bh, g8), jnp.float32),
        pltpu.VMEM((tc * bh, g8), jnp.float32),
        pltpu.VMEM((bh, hid2), jnp.bfloat16),
        pltpu.VMEM((bh, hid2), jnp.float32),
        pltpu.VMEM((bh, hid2), jnp.float32),
    ]
    out = pl.pallas_call(
        functools.partial(_lstm_body, tc, bh, hid2, n_in, accumulate, T, nt),
        grid=(nb, nt),
        in_specs=in_specs,
        out_specs=out_specs,
        out_shape=out_shape,
        scratch_shapes=scratch,
        compiler_params=pltpu.CompilerParams(
            dimension_semantics=("parallel", "arbitrary"),
            vmem_limit_bytes=56 * 1024 * 1024),
    )(*args)
    return out


def _spread(w, slot, hid):
    z = jnp.zeros(w.shape[:-1] + (hid,), w.dtype)
    parts = []
    for q in range(4):
        blk = w[..., q * hid:(q + 1) * hid]
        parts.extend([blk, z] if slot == 0 else [z, blk])
    return jnp.concatenate(parts, axis=-1)


def kernel(x, binpoints, w_ih_0, w_hh_0, b_ih_0, b_hh_0,
           w_ih_1, w_hh_1, b_ih_1, b_hh_1,
           w_ih_2, w_hh_2, b_ih_2, b_hh_2):
    B, T, NB = x.shape
    hid = w_hh_0.shape[-1]
    bh = B // 2
    tcf = 200 if T % 200 == 0 else T
    tc = 100 if T % 100 == 0 else T

    h0 = _frontend(x, binpoints, bh, tcf)
    h0t = jnp.transpose(h0, (1, 0, 2))

    def wiT(w):
        return jnp.transpose(w, (0, 2, 1))

    wi0T = jnp.pad(wiT(w_ih_0), ((0, 0), (0, _FPAD - _NFILT), (0, 0)))
    wi1T, wi2T = wiT(w_ih_1), wiT(w_ih_2)
    wh0T, wh1T, wh2T = wiT(w_hh_0), wiT(w_hh_1), wiT(w_hh_2)

    def combine_wh(whT):
        return jnp.concatenate(
            [_spread(whT[0], 0, hid), _spread(whT[1], 1, hid)],
            axis=0).astype(jnp.bfloat16)

    def biases(b_ih, b_hh):
        bs = b_ih + b_hh
        return (_spread(bs[0].reshape(1, -1), 0, hid),
                _spread(bs[1].reshape(1, -1), 1, hid))

    def zero_rows(w, keep_top):
        top, bot = w[:hid], w[hid:]
        if keep_top:
            return jnp.concatenate([top, jnp.zeros_like(bot)], axis=0)
        return jnp.concatenate([jnp.zeros_like(top), bot], axis=0)

    bf0, bb0 = biases(b_ih_0, b_hh_0)
    f0, r0 = _lstm_layer(
        [h0t],
        [_spread(wi0T[0], 0, hid).astype(jnp.bfloat16)],
        [_spread(wi0T[1], 1, hid).astype(jnp.bfloat16)],
        combine_wh(wh0T), bf0, bb0, bh, tc, False)

    def mk_io_weights(wT):
        wfs = [_spread(zero_rows(wT[0], True), 0, hid).astype(jnp.bfloat16),
               _spread(zero_rows(wT[0], False), 0, hid).astype(jnp.bfloat16)]
        wbs = [_spread(zero_rows(wT[1], True), 1, hid).astype(jnp.bfloat16),
               _spread(zero_rows(wT[1], False), 1, hid).astype(jnp.bfloat16)]
        return wfs, wbs

    wfs1, wbs1 = mk_io_weights(wi1T)
    bf1, bb1 = biases(b_ih_1, b_hh_1)
    f1, r1 = _lstm_layer([f0, r0], wfs1, wbs1, combine_wh(wh1T),
                         bf1, bb1, bh, tc, False)

    wfs2, wbs2 = mk_io_weights(wi2T)
    bf2, bb2 = biases(b_ih_2, b_hh_2)
    (mean_out,) = _lstm_layer([f1, r1], wfs2, wbs2, combine_wh(wh2T),
                              bf2, bb2, bh, tc, True)
    return mean_out


# speedup vs baseline: 1.2259x; 1.2259x over previous
import functools

import jax
import jax.numpy as jnp
from jax.experimental import pallas as pl
from jax.experimental.pallas import tpu as pltpu

_NFILT = 40
_FPAD = 128


def _frontend_body(nfilt, b0_ref, b1_ref, b2_ref, x_ref, o_ref):
    nb = x_ref.shape[-1]
    fp = o_ref.shape[-1]
    b0, b1, b2 = b0_ref[...], b1_ref[...], b2_ref[...]
    f0, f1, f2 = jnp.floor(b0), jnp.floor(b1), jnp.floor(b2)
    i = jax.lax.broadcasted_iota(jnp.int32, (nb, fp), 0).astype(jnp.float32)
    j = jax.lax.broadcasted_iota(jnp.int32, (nb, fp), 1)
    rise_m = (i >= f0) & (i < f1)
    fall_m = (i >= f1) & (i < f2)
    d1 = b1 - b0
    d2 = b2 - b1
    rv = (i - b0) / jnp.where(d1 > 0, d1, 1.0) ** 2
    fv = (b2 - i) / jnp.where(d2 > 0, d2, 1.0) ** 2
    val = jnp.where(fall_m, fv, jnp.where(rise_m, rv, 0.0))
    val = jnp.where(j < nfilt - 1, val, 0.0)
    val = jnp.where(j == 0, jnp.where(i == 0.0, 1.0, 0.0), val)

    bh, tc, _ = x_ref.shape
    xb = x_ref[...].reshape(bh * tc, nb)
    filt = jnp.dot(xb, val, preferred_element_type=jnp.float32)
    h = jnp.log(filt + 1e-10).astype(jnp.bfloat16)
    o_ref[...] = h.reshape(bh, tc, fp)


def _frontend(x, binpoints, bh, tcf):
    B, T, NB = x.shape
    nt = T // tcf
    pad = _FPAD - _NFILT
    b0 = jnp.pad(binpoints[0:_NFILT], (0, pad)).reshape(1, _FPAD)
    b1 = jnp.pad(binpoints[1:_NFILT + 1], (0, pad)).reshape(1, _FPAD)
    b2 = jnp.pad(binpoints[2:_NFILT + 2], (0, pad)).reshape(1, _FPAD)
    return pl.pallas_call(
        functools.partial(_frontend_body, _NFILT),
        grid=(B // bh, nt),
        in_specs=[
            pl.BlockSpec((1, _FPAD), lambda b, k: (0, 0)),
            pl.BlockSpec((1, _FPAD), lambda b, k: (0, 0)),
            pl.BlockSpec((1, _FPAD), lambda b, k: (0, 0)),
            pl.BlockSpec((bh, tcf, NB), lambda b, k: (b, k, 0)),
        ],
        out_specs=pl.BlockSpec((bh, tcf, _FPAD), lambda b, k: (b, k, 0)),
        out_shape=jax.ShapeDtypeStruct((B, T, _FPAD), jnp.bfloat16),
        compiler_params=pltpu.CompilerParams(
            dimension_semantics=("parallel", "arbitrary")),
    )(b0, b1, b2, x)


def _lstm_body(tc, bh, hid2, n_in, accumulate, t_total, nt, *refs):
    g8 = 4 * hid2
    xf = refs[0:n_in]
    xb = refs[n_in:2 * n_in]
    wpf, wpb, wc, bf, bb = refs[2 * n_in:2 * n_in + 5]
    n_out = 1 if accumulate else 2
    outs = refs[2 * n_in + 5:2 * n_in + 5 + n_out]
    pf_s, pb_s, h_s, c_s, a_s = refs[2 * n_in + 5 + n_out:]
    k = pl.program_id(1)
    rows = tc * bh
    mtc = 1
    for cand in range(tc, 0, -1):
        if tc % cand == 0 and cand * bh <= 1024:
            mtc = cand
            break
    mchunk = mtc * bh

    @pl.when(k == 0)
    def _():
        h_s[...] = jnp.zeros_like(h_s)
        c_s[...] = jnp.zeros_like(c_s)
        a_s[...] = jnp.zeros_like(a_s)

    def compute_pre(xs, w_ref, b_ref, out_ref):
        w = w_ref[...]
        pltpu.matmul_push_rhs(w[:, 0:256], 1, 0)
        pltpu.matmul_push_rhs(w[:, 256:512], 1, 1)
        bv = b_ref[...]
        for mt in range(0, tc, mtc):
            parts = [xr[pl.ds(mt, mtc), :, :].reshape(mchunk, xr.shape[-1])
                     for xr in xs]
            if len(parts) == 1:
                parts.append(jnp.zeros(
                    (mchunk, 256 - parts[0].shape[-1]), jnp.bfloat16))
            lhs = jnp.concatenate(parts, axis=1)
            lsr = 1 if mt == 0 else None
            pltpu.matmul_acc_lhs(0, lhs, 0, load_staged_rhs=lsr)
            pltpu.matmul_acc_lhs(0, lhs, 1, load_staged_rhs=lsr)
            p0 = pltpu.matmul_pop(0, (mchunk, 256), jnp.float32, 0)
            p1 = pltpu.matmul_pop(0, (mchunk, 256), jnp.float32, 1)
            m = mt * bh
            out_ref[m:m + mchunk, 0:256] = p0 + bv[:, 0:256]
            out_ref[m:m + mchunk, 256:512] = p1 + bv[:, 256:512]

    compute_pre(xf, wpf, bf, pf_s)
    compute_pre(xb, wpb, bb, pb_s)

    wc_v = wc[...]
    ns = 1
    sb = bh // ns
    zpad = jnp.zeros((sb, 128), jnp.bfloat16)
    pltpu.matmul_push_rhs(wc_v[:, 0:256], 0, 0)
    pltpu.matmul_push_rhs(wc_v[:, 256:512], 0, 1)
    zlatch = jnp.zeros((16, 256), jnp.bfloat16)
    pltpu.matmul_acc_lhs(0, zlatch, 0, load_staged_rhs=0)
    pltpu.matmul_acc_lhs(0, zlatch, 1, load_staged_rhs=0)
    _d0 = pltpu.matmul_pop(0, (16, 256), jnp.float32, 0)
    _d1 = pltpu.matmul_pop(0, (16, 256), jnp.float32, 1)

    def step(t, carry):
        hs, cs, acs = carry
        tb = tc - 1 - t
        new_h, new_c, new_a, new_hf = [], [], [], []
        for s in range(ns):
            rf = pl.multiple_of(t * bh + s * sb, sb)
            rb = pl.multiple_of(tb * bh + s * sb, sb)
            hp = jnp.concatenate([hs[s], zpad], axis=1)
            pltpu.matmul_acc_lhs(0, hp, 0)
            pltpu.matmul_acc_lhs(0, hp, 1)
            m0 = pltpu.matmul_pop(0, (sb, 256), jnp.float32, 0)
            m1 = pltpu.matmul_pop(0, (sb, 256), jnp.float32, 1)
            g = (pf_s[pl.ds(rf, sb), :] + pb_s[pl.ds(rb, sb), :]
                 + jnp.concatenate([m0, m1], axis=1))
            ig = jax.nn.sigmoid(g[:, 0:hid2])
            fg = jax.nn.sigmoid(g[:, hid2:2 * hid2])
            gg = jnp.tanh(g[:, 2 * hid2:3 * hid2])
            og = jax.nn.sigmoid(g[:, 3 * hid2:4 * hid2])
            c = fg * cs[s] + ig * gg
            hf32 = og * jnp.tanh(c)
            new_c.append(c)
            new_hf.append(hf32)
            new_h.append(hf32.astype(jnp.bfloat16))
            if accumulate:
                new_a.append(acs[s] + hf32)
        if not accumulate:
            hcat = jnp.concatenate(new_h, axis=0)
            outs[0][pl.ds(t, 1)] = hcat.reshape(1, bh, hid2)
            outs[1][pl.ds(tb, 1)] = hcat.reshape(1, bh, hid2)
            new_a = list(acs)
        return (tuple(new_h), tuple(new_c), tuple(new_a))

    init = (tuple(h_s[pl.ds(s * sb, sb), :] for s in range(ns)),
            tuple(c_s[pl.ds(s * sb, sb), :] for s in range(ns)),
            tuple(a_s[pl.ds(s * sb, sb), :] for s in range(ns)))
    fin = jax.lax.fori_loop(0, tc, step, init)
    h_s[...] = jnp.concatenate(fin[0], axis=0)
    c_s[...] = jnp.concatenate(fin[1], axis=0)
    a_s[...] = jnp.concatenate(fin[2], axis=0)

    if accumulate:
        @pl.when(k == nt - 1)
        def _():
            outs[0][...] = jnp.concatenate(fin[2], axis=0) * (1.0 / t_total)


def _lstm_layer(ins, wpf, wpb, wc, bf, bb, bh, tc, accumulate):
    T, B, _ = ins[0].shape
    g8 = wc.shape[1]
    hid2 = g8 // 4
    nt = T // tc
    nb = B // bh
    n_in = len(ins)

    in_specs = []
    args = []
    for xr in ins:
        d = xr.shape[-1]
        in_specs.append(pl.BlockSpec((tc, bh, d), lambda b, k: (k, b, 0)))
        args.append(xr)
    for xr in ins:
        d = xr.shape[-1]
        in_specs.append(
            pl.BlockSpec((tc, bh, d), lambda b, k: (nt - 1 - k, b, 0)))
        args.append(xr)
    for w in (wpf, wpb, wc, bf, bb):
        in_specs.append(pl.BlockSpec(w.shape, lambda b, k: (0,) * w.ndim))
        args.append(w)

    if accumulate:
        out_shape = (jax.ShapeDtypeStruct((B, hid2), jnp.float32),)
        out_specs = [pl.BlockSpec((bh, hid2), lambda b, k: (b, 0))]
    else:
        out_shape = (jax.ShapeDtypeStruct((T, B, hid2), jnp.bfloat16),) * 2
        out_specs = [
            pl.BlockSpec((tc, bh, hid2), lambda b, k: (k, b, 0)),
            pl.BlockSpec((tc, bh, hid2), lambda b, k: (nt - 1 - k, b, 0)),
        ]

    scratch = [
        pltpu.VMEM((tc * bh, g8), jnp.float32),
        pltpu.VMEM((tc * bh, g8), jnp.float32),
        pltpu.VMEM((bh, hid2), jnp.bfloat16),
        pltpu.VMEM((bh, hid2), jnp.float32),
        pltpu.VMEM((bh, hid2), jnp.float32),
    ]
    out = pl.pallas_call(
        functools.partial(_lstm_body, tc, bh, hid2, n_in, accumulate, T, nt),
        grid=(nb, nt),
        in_specs=in_specs,
        out_specs=out_specs,
        out_shape=out_shape,
        scratch_shapes=scratch,
        compiler_params=pltpu.CompilerParams(
            dimension_semantics=("parallel", "arbitrary"),
            vmem_limit_bytes=56 * 1024 * 1024),
    )(*args)
    return out


def _spread(w, slot, hid):
    z = jnp.zeros(w.shape[:-1] + (hid,), w.dtype)
    parts = []
    for q in range(4):
        blk = w[..., q * hid:(q + 1) * hid]
        parts.extend([blk, z] if slot == 0 else [z, blk])
    return jnp.concatenate(parts, axis=-1)


def kernel(x, binpoints, w_ih_0, w_hh_0, b_ih_0, b_hh_0,
           w_ih_1, w_hh_1, b_ih_1, b_hh_1,
           w_ih_2, w_hh_2, b_ih_2, b_hh_2):
    B, T, NB = x.shape
    hid = w_hh_0.shape[-1]
    bh = B // 2
    tcf = 200 if T % 200 == 0 else T
    tc = 100 if T % 100 == 0 else T

    h0 = _frontend(x, binpoints, bh, tcf)
    h0t = jnp.transpose(h0, (1, 0, 2))

    def wiT(w):
        return jnp.transpose(w, (0, 2, 1))

    wi0T = jnp.pad(wiT(w_ih_0), ((0, 0), (0, _FPAD - _NFILT), (0, 0)))
    wi1T, wi2T = wiT(w_ih_1), wiT(w_ih_2)
    wh0T, wh1T, wh2T = wiT(w_hh_0), wiT(w_hh_1), wiT(w_hh_2)

    def combine_wh(whT):
        wc = jnp.concatenate(
            [_spread(whT[0], 0, hid), _spread(whT[1], 1, hid)], axis=0)
        return jnp.pad(wc, ((0, 256 - wc.shape[0]), (0, 0))).astype(jnp.bfloat16)

    def biases(b_ih, b_hh):
        bs = b_ih + b_hh
        return (_spread(bs[0].reshape(1, -1), 0, hid),
                _spread(bs[1].reshape(1, -1), 1, hid))

    def zero_rows(w, keep_top):
        top, bot = w[:hid], w[hid:]
        if keep_top:
            return jnp.concatenate([top, jnp.zeros_like(bot)], axis=0)
        return jnp.concatenate([jnp.zeros_like(top), bot], axis=0)

    def stack256(w_top, w_bot):
        return jnp.concatenate([w_top, w_bot], axis=0).astype(jnp.bfloat16)

    bf0, bb0 = biases(b_ih_0, b_hh_0)
    z128 = jnp.zeros((128, 512), jnp.float32)
    f0, r0 = _lstm_layer(
        [h0t],
        stack256(_spread(wi0T[0], 0, hid), z128),
        stack256(_spread(wi0T[1], 1, hid), z128),
        combine_wh(wh0T), bf0, bb0, bh, tc, False)

    def mk_io_weights(wT):
        wpf = stack256(_spread(zero_rows(wT[0], True), 0, hid),
                       _spread(zero_rows(wT[0], False), 0, hid))
        wpb = stack256(_spread(zero_rows(wT[1], True), 1, hid),
                       _spread(zero_rows(wT[1], False), 1, hid))
        return wpf, wpb

    wfs1, wbs1 = mk_io_weights(wi1T)
    bf1, bb1 = biases(b_ih_1, b_hh_1)
    f1, r1 = _lstm_layer([f0, r0], wfs1, wbs1, combine_wh(wh1T),
                         bf1, bb1, bh, tc, False)

    wfs2, wbs2 = mk_io_weights(wi2T)
    bf2, bb2 = biases(b_ih_2, b_hh_2)
    (mean_out,) = _lstm_layer([f1, r1], wfs2, wbs2, combine_wh(wh2T),
                              bf2, bb2, bh, tc, True)
    return mean_out


# speedup vs baseline: 1.2591x; 1.0271x over previous
import functools

import jax
import jax.numpy as jnp
from jax.experimental import pallas as pl
from jax.experimental.pallas import tpu as pltpu

_NFILT = 40
_FPAD = 128


def _frontend_body(nfilt, b0_ref, b1_ref, b2_ref, x_ref, o_ref):
    nb = x_ref.shape[-1]
    fp = o_ref.shape[-1]
    b0, b1, b2 = b0_ref[...], b1_ref[...], b2_ref[...]
    f0, f1, f2 = jnp.floor(b0), jnp.floor(b1), jnp.floor(b2)
    i = jax.lax.broadcasted_iota(jnp.int32, (nb, fp), 0).astype(jnp.float32)
    j = jax.lax.broadcasted_iota(jnp.int32, (nb, fp), 1)
    rise_m = (i >= f0) & (i < f1)
    fall_m = (i >= f1) & (i < f2)
    d1 = b1 - b0
    d2 = b2 - b1
    rv = (i - b0) / jnp.where(d1 > 0, d1, 1.0) ** 2
    fv = (b2 - i) / jnp.where(d2 > 0, d2, 1.0) ** 2
    val = jnp.where(fall_m, fv, jnp.where(rise_m, rv, 0.0))
    val = jnp.where(j < nfilt - 1, val, 0.0)
    val = jnp.where(j == 0, jnp.where(i == 0.0, 1.0, 0.0), val)

    bh, tc, _ = x_ref.shape
    xb = x_ref[...].reshape(bh * tc, nb)
    filt = jnp.dot(xb, val, preferred_element_type=jnp.float32)
    h = jnp.log(filt + 1e-10).astype(jnp.bfloat16)
    o_ref[...] = h.reshape(bh, tc, fp)


def _frontend(x, binpoints, bh, tcf):
    B, T, NB = x.shape
    nt = T // tcf
    pad = _FPAD - _NFILT
    b0 = jnp.pad(binpoints[0:_NFILT], (0, pad)).reshape(1, _FPAD)
    b1 = jnp.pad(binpoints[1:_NFILT + 1], (0, pad)).reshape(1, _FPAD)
    b2 = jnp.pad(binpoints[2:_NFILT + 2], (0, pad)).reshape(1, _FPAD)
    return pl.pallas_call(
        functools.partial(_frontend_body, _NFILT),
        grid=(B // bh, nt),
        in_specs=[
            pl.BlockSpec((1, _FPAD), lambda b, k: (0, 0)),
            pl.BlockSpec((1, _FPAD), lambda b, k: (0, 0)),
            pl.BlockSpec((1, _FPAD), lambda b, k: (0, 0)),
            pl.BlockSpec((bh, tcf, NB), lambda b, k: (b, k, 0)),
        ],
        out_specs=pl.BlockSpec((bh, tcf, _FPAD), lambda b, k: (b, k, 0)),
        out_shape=jax.ShapeDtypeStruct((B, T, _FPAD), jnp.bfloat16),
        compiler_params=pltpu.CompilerParams(
            dimension_semantics=("parallel", "arbitrary")),
    )(b0, b1, b2, x)


def _lstm_body(tc, bh, hid2, n_in, accumulate, t_total, nt, *refs):
    g8 = 4 * hid2
    xf = refs[0:n_in]
    xb = refs[n_in:2 * n_in]
    wpf, wpb, wc, bf, bb = refs[2 * n_in:2 * n_in + 5]
    n_out = 1 if accumulate else 2
    outs = refs[2 * n_in + 5:2 * n_in + 5 + n_out]
    pf_s, pb_s, h_s, c_s, a_s = refs[2 * n_in + 5 + n_out:]
    k = pl.program_id(1)
    rows = tc * bh
    mtc = 1
    for cand in range(tc, 0, -1):
        if tc % cand == 0 and cand * bh <= 1024:
            mtc = cand
            break
    mchunk = mtc * bh

    @pl.when(k == 0)
    def _():
        h_s[...] = jnp.zeros_like(h_s)
        c_s[...] = jnp.zeros_like(c_s)
        a_s[...] = jnp.zeros_like(a_s)

    def compute_pre(xs, w_ref, b_ref, out_ref):
        w = w_ref[...]
        pltpu.matmul_push_rhs(w[:, 0:256], 1, 0)
        pltpu.matmul_push_rhs(w[:, 256:512], 1, 1)
        bv = b_ref[...]
        for mt in range(0, tc, mtc):
            parts = [xr[pl.ds(mt, mtc), :, :].reshape(mchunk, xr.shape[-1])
                     for xr in xs]
            if len(parts) == 1:
                parts.append(jnp.zeros(
                    (mchunk, 256 - parts[0].shape[-1]), jnp.bfloat16))
            lhs = jnp.concatenate(parts, axis=1)
            lsr = 1 if mt == 0 else None
            pltpu.matmul_acc_lhs(0, lhs, 0, load_staged_rhs=lsr)
            pltpu.matmul_acc_lhs(0, lhs, 1, load_staged_rhs=lsr)
            p0 = pltpu.matmul_pop(0, (mchunk, 256), jnp.float32, 0)
            p1 = pltpu.matmul_pop(0, (mchunk, 256), jnp.float32, 1)
            m = mt * bh
            out_ref[m:m + mchunk, 0:256] = p0 + bv[:, 0:256]
            out_ref[m:m + mchunk, 256:512] = p1 + bv[:, 256:512]

    compute_pre(xf, wpf, bf, pf_s)
    compute_pre(xb, wpb, bb, pb_s)

    wc_v = wc[...]
    ns = 1
    sb = bh // ns
    zpad = jnp.zeros((sb, 128), jnp.bfloat16)
    pltpu.matmul_push_rhs(wc_v[:, 0:256], 0, 0)
    pltpu.matmul_push_rhs(wc_v[:, 256:512], 0, 1)
    zlatch = jnp.zeros((16, 256), jnp.bfloat16)
    pltpu.matmul_acc_lhs(0, zlatch, 0, load_staged_rhs=0)
    pltpu.matmul_acc_lhs(0, zlatch, 1, load_staged_rhs=0)
    _d0 = pltpu.matmul_pop(0, (16, 256), jnp.float32, 0)
    _d1 = pltpu.matmul_pop(0, (16, 256), jnp.float32, 1)

    def step(t, carry):
        hs, cs, acs = carry
        tb = tc - 1 - t
        new_h, new_c, new_a, new_hf = [], [], [], []
        for s in range(ns):
            rf = pl.multiple_of(t * bh + s * sb, sb)
            rb = pl.multiple_of(tb * bh + s * sb, sb)
            hp = jnp.concatenate([hs[s], zpad], axis=1)
            pltpu.matmul_acc_lhs(0, hp, 0)
            pltpu.matmul_acc_lhs(0, hp, 1)
            m0 = pltpu.matmul_pop(0, (sb, 256), jnp.float32, 0)
            m1 = pltpu.matmul_pop(0, (sb, 256), jnp.float32, 1)
            g = (pf_s[pl.ds(rf, sb), :] + pb_s[pl.ds(rb, sb), :]
                 + jnp.concatenate([m0, m1], axis=1))
            tg = jnp.tanh(g)
            sif = tg[:, 0:2 * hid2] * 0.5 + 0.5
            ig = sif[:, 0:hid2]
            fg = sif[:, hid2:2 * hid2]
            gg = tg[:, 2 * hid2:3 * hid2]
            og = tg[:, 3 * hid2:4 * hid2] * 0.5 + 0.5
            c = fg * cs[s] + ig * gg
            hf32 = og * jnp.tanh(c)
            new_c.append(c)
            new_hf.append(hf32)
            new_h.append(hf32.astype(jnp.bfloat16))
            if accumulate:
                new_a.append(acs[s] + hf32)
        if not accumulate:
            hcat = jnp.concatenate(new_h, axis=0)
            outs[0][pl.ds(t, 1)] = hcat.reshape(1, bh, hid2)
            outs[1][pl.ds(tb, 1)] = hcat.reshape(1, bh, hid2)
            new_a = list(acs)
        return (tuple(new_h), tuple(new_c), tuple(new_a))

    init = (tuple(h_s[pl.ds(s * sb, sb), :] for s in range(ns)),
            tuple(c_s[pl.ds(s * sb, sb), :] for s in range(ns)),
            tuple(a_s[pl.ds(s * sb, sb), :] for s in range(ns)))
    fin = jax.lax.fori_loop(0, tc, step, init)
    h_s[...] = jnp.concatenate(fin[0], axis=0)
    c_s[...] = jnp.concatenate(fin[1], axis=0)
    a_s[...] = jnp.concatenate(fin[2], axis=0)

    if accumulate:
        @pl.when(k == nt - 1)
        def _():
            outs[0][...] = jnp.concatenate(fin[2], axis=0) * (1.0 / t_total)


def _lstm_layer(ins, wpf, wpb, wc, bf, bb, bh, tc, accumulate):
    T, B, _ = ins[0].shape
    g8 = wc.shape[1]
    hid2 = g8 // 4
    nt = T // tc
    nb = B // bh
    n_in = len(ins)

    in_specs = []
    args = []
    for xr in ins:
        d = xr.shape[-1]
        in_specs.append(pl.BlockSpec((tc, bh, d), lambda b, k: (k, b, 0)))
        args.append(xr)
    for xr in ins:
        d = xr.shape[-1]
        in_specs.append(
            pl.BlockSpec((tc, bh, d), lambda b, k: (nt - 1 - k, b, 0)))
        args.append(xr)
    for w in (wpf, wpb, wc, bf, bb):
        in_specs.append(pl.BlockSpec(w.shape, lambda b, k: (0,) * w.ndim))
        args.append(w)

    if accumulate:
        out_shape = (jax.ShapeDtypeStruct((B, hid2), jnp.float32),)
        out_specs = [pl.BlockSpec((bh, hid2), lambda b, k: (b, 0))]
    else:
        out_shape = (jax.ShapeDtypeStruct((T, B, hid2), jnp.bfloat16),) * 2
        out_specs = [
            pl.BlockSpec((tc, bh, hid2), lambda b, k: (k, b, 0)),
            pl.BlockSpec((tc, bh, hid2), lambda b, k: (nt - 1 - k, b, 0)),
        ]

    scratch = [
        pltpu.VMEM((tc * bh, g8), jnp.float32),
        pltpu.VMEM((tc * bh, g8), jnp.float32),
        pltpu.VMEM((bh, hid2), jnp.bfloat16),
        pltpu.VMEM((bh, hid2), jnp.float32),
        pltpu.VMEM((bh, hid2), jnp.float32),
    ]
    out = pl.pallas_call(
        functools.partial(_lstm_body, tc, bh, hid2, n_in, accumulate, T, nt),
        grid=(nb, nt),
        in_specs=in_specs,
        out_specs=out_specs,
        out_shape=out_shape,
        scratch_shapes=scratch,
        compiler_params=pltpu.CompilerParams(
            dimension_semantics=("parallel", "arbitrary"),
            vmem_limit_bytes=56 * 1024 * 1024),
    )(*args)
    return out


def _spread(w, slot, hid):
    z = jnp.zeros(w.shape[:-1] + (hid,), w.dtype)
    parts = []
    for q in range(4):
        blk = w[..., q * hid:(q + 1) * hid]
        if q != 2:
            blk = blk * 0.5
        parts.extend([blk, z] if slot == 0 else [z, blk])
    return jnp.concatenate(parts, axis=-1)


def kernel(x, binpoints, w_ih_0, w_hh_0, b_ih_0, b_hh_0,
           w_ih_1, w_hh_1, b_ih_1, b_hh_1,
           w_ih_2, w_hh_2, b_ih_2, b_hh_2):
    B, T, NB = x.shape
    hid = w_hh_0.shape[-1]
    bh = B // 2
    tcf = 200 if T % 200 == 0 else T
    tc = 100 if T % 100 == 0 else T

    h0 = _frontend(x, binpoints, bh, tcf)
    h0t = jnp.transpose(h0, (1, 0, 2))

    def wiT(w):
        return jnp.transpose(w, (0, 2, 1))

    wi0T = jnp.pad(wiT(w_ih_0), ((0, 0), (0, _FPAD - _NFILT), (0, 0)))
    wi1T, wi2T = wiT(w_ih_1), wiT(w_ih_2)
    wh0T, wh1T, wh2T = wiT(w_hh_0), wiT(w_hh_1), wiT(w_hh_2)

    def combine_wh(whT):
        wc = jnp.concatenate(
            [_spread(whT[0], 0, hid), _spread(whT[1], 1, hid)], axis=0)
        return jnp.pad(wc, ((0, 256 - wc.shape[0]), (0, 0))).astype(jnp.bfloat16)

    def biases(b_ih, b_hh):
        bs = b_ih + b_hh
        return (_spread(bs[0].reshape(1, -1), 0, hid),
                _spread(bs[1].reshape(1, -1), 1, hid))

    def zero_rows(w, keep_top):
        top, bot = w[:hid], w[hid:]
        if keep_top:
            return jnp.concatenate([top, jnp.zeros_like(bot)], axis=0)
        return jnp.concatenate([jnp.zeros_like(top), bot], axis=0)

    def stack256(w_top, w_bot):
        return jnp.concatenate([w_top, w_bot], axis=0).astype(jnp.bfloat16)

    bf0, bb0 = biases(b_ih_0, b_hh_0)
    z128 = jnp.zeros((128, 512), jnp.float32)
    f0, r0 = _lstm_layer(
        [h0t],
        stack256(_spread(wi0T[0], 0, hid), z128),
        stack256(_spread(wi0T[1], 1, hid), z128),
        combine_wh(wh0T), bf0, bb0, bh, tc, False)

    def mk_io_weights(wT):
        wpf = stack256(_spread(zero_rows(wT[0], True), 0, hid),
                       _spread(zero_rows(wT[0], False), 0, hid))
        wpb = stack256(_spread(zero_rows(wT[1], True), 1, hid),
                       _spread(zero_rows(wT[1], False), 1, hid))
        return wpf, wpb

    wfs1, wbs1 = mk_io_weights(wi1T)
    bf1, bb1 = biases(b_ih_1, b_hh_1)
    f1, r1 = _lstm_layer([f0, r0], wfs1, wbs1, combine_wh(wh1T),
                         bf1, bb1, bh, tc, False)

    wfs2, wbs2 = mk_io_weights(wi2T)
    bf2, bb2 = biases(b_ih_2, b_hh_2)
    (mean_out,) = _lstm_layer([f1, r1], wfs2, wbs2, combine_wh(wh2T),
                              bf2, bb2, bh, tc, True)
    return mean_out


# speedup vs baseline: 1.2609x; 1.0014x over previous
import functools

import jax
import jax.numpy as jnp
from jax.experimental import pallas as pl
from jax.experimental.pallas import tpu as pltpu

_NFILT = 40
_FPAD = 128


def _frontend_body(nfilt, b0_ref, b1_ref, b2_ref, x_ref, o_ref):
    nb = x_ref.shape[-1]
    fp = o_ref.shape[-1]
    b0, b1, b2 = b0_ref[...], b1_ref[...], b2_ref[...]
    f0, f1, f2 = jnp.floor(b0), jnp.floor(b1), jnp.floor(b2)
    i = jax.lax.broadcasted_iota(jnp.int32, (nb, fp), 0).astype(jnp.float32)
    j = jax.lax.broadcasted_iota(jnp.int32, (nb, fp), 1)
    rise_m = (i >= f0) & (i < f1)
    fall_m = (i >= f1) & (i < f2)
    d1 = b1 - b0
    d2 = b2 - b1
    rv = (i - b0) / jnp.where(d1 > 0, d1, 1.0) ** 2
    fv = (b2 - i) / jnp.where(d2 > 0, d2, 1.0) ** 2
    val = jnp.where(fall_m, fv, jnp.where(rise_m, rv, 0.0))
    val = jnp.where(j < nfilt - 1, val, 0.0)
    val = jnp.where(j == 0, jnp.where(i == 0.0, 1.0, 0.0), val)

    bh, tc, _ = x_ref.shape
    xb = x_ref[...].reshape(bh * tc, nb)
    filt = jnp.dot(xb, val, preferred_element_type=jnp.float32)
    h = jnp.log(filt + 1e-10).astype(jnp.bfloat16)
    o_ref[...] = h.reshape(bh, tc, fp)


def _frontend(x, binpoints, bh, tcf):
    B, T, NB = x.shape
    nt = T // tcf
    pad = _FPAD - _NFILT
    b0 = jnp.pad(binpoints[0:_NFILT], (0, pad)).reshape(1, _FPAD)
    b1 = jnp.pad(binpoints[1:_NFILT + 1], (0, pad)).reshape(1, _FPAD)
    b2 = jnp.pad(binpoints[2:_NFILT + 2], (0, pad)).reshape(1, _FPAD)
    return pl.pallas_call(
        functools.partial(_frontend_body, _NFILT),
        grid=(B // bh, nt),
        in_specs=[
            pl.BlockSpec((1, _FPAD), lambda b, k: (0, 0)),
            pl.BlockSpec((1, _FPAD), lambda b, k: (0, 0)),
            pl.BlockSpec((1, _FPAD), lambda b, k: (0, 0)),
            pl.BlockSpec((bh, tcf, NB), lambda b, k: (b, k, 0)),
        ],
        out_specs=pl.BlockSpec((bh, tcf, _FPAD), lambda b, k: (b, k, 0)),
        out_shape=jax.ShapeDtypeStruct((B, T, _FPAD), jnp.bfloat16),
        compiler_params=pltpu.CompilerParams(
            dimension_semantics=("parallel", "arbitrary")),
    )(b0, b1, b2, x)


def _lstm_body(tc, bh, hid2, n_in, accumulate, t_total, nt, *refs):
    g8 = 4 * hid2
    xf = refs[0:n_in]
    xb = refs[n_in:2 * n_in]
    wpf, wpb, wc, bf, bb = refs[2 * n_in:2 * n_in + 5]
    n_out = 1 if accumulate else 2
    outs = refs[2 * n_in + 5:2 * n_in + 5 + n_out]
    pf_s, pb_s, h_s, c_s, a_s = refs[2 * n_in + 5 + n_out:]
    k = pl.program_id(1)
    rows = tc * bh
    mtc = 1
    for cand in range(tc, 0, -1):
        if tc % cand == 0 and cand * bh <= 1024:
            mtc = cand
            break
    mchunk = mtc * bh

    @pl.when(k == 0)
    def _():
        h_s[...] = jnp.zeros_like(h_s)
        c_s[...] = jnp.zeros_like(c_s)
        a_s[...] = jnp.zeros_like(a_s)

    def compute_pre(xs, w_ref, b_ref, out_ref):
        w = w_ref[...]
        pltpu.matmul_push_rhs(w[:, 0:256], 1, 0)
        pltpu.matmul_push_rhs(w[:, 256:512], 1, 1)
        bv = b_ref[...]
        for mt in range(0, tc, mtc):
            parts = [xr[pl.ds(mt, mtc), :, :].reshape(mchunk, xr.shape[-1])
                     for xr in xs]
            if len(parts) == 1:
                parts.append(jnp.zeros(
                    (mchunk, 256 - parts[0].shape[-1]), jnp.bfloat16))
            lhs = jnp.concatenate(parts, axis=1)
            lsr = 1 if mt == 0 else None
            pltpu.matmul_acc_lhs(0, lhs, 0, load_staged_rhs=lsr)
            pltpu.matmul_acc_lhs(0, lhs, 1, load_staged_rhs=lsr)
            p0 = pltpu.matmul_pop(0, (mchunk, 256), jnp.float32, 0)
            p1 = pltpu.matmul_pop(0, (mchunk, 256), jnp.float32, 1)
            m = mt * bh
            out_ref[m:m + mchunk, 0:256] = p0 + bv[:, 0:256]
            out_ref[m:m + mchunk, 256:512] = p1 + bv[:, 256:512]

    compute_pre(xf, wpf, bf, pf_s)
    compute_pre(xb, wpb, bb, pb_s)

    wc_v = wc[...]
    ns = 2
    sb = bh // ns
    zpad = jnp.zeros((sb, 128), jnp.bfloat16)
    pltpu.matmul_push_rhs(wc_v[:, 0:256], 0, 0)
    pltpu.matmul_push_rhs(wc_v[:, 256:512], 0, 1)
    zlatch = jnp.zeros((16, 256), jnp.bfloat16)
    pltpu.matmul_acc_lhs(0, zlatch, 0, load_staged_rhs=0)
    pltpu.matmul_acc_lhs(0, zlatch, 1, load_staged_rhs=0)
    _d0 = pltpu.matmul_pop(0, (16, 256), jnp.float32, 0)
    _d1 = pltpu.matmul_pop(0, (16, 256), jnp.float32, 1)

    def step(t, carry):
        hs, cs, acs = carry
        tb = tc - 1 - t
        new_h, new_c, new_a, new_hf = [], [], [], []
        for s in range(ns):
            hp = jnp.concatenate([hs[s], zpad], axis=1)
            pltpu.matmul_acc_lhs(s * 8, hp, 0)
            pltpu.matmul_acc_lhs(s * 8, hp, 1)
        for s in range(ns):
            rf = pl.multiple_of(t * bh + s * sb, sb)
            rb = pl.multiple_of(tb * bh + s * sb, sb)
            m0 = pltpu.matmul_pop(s * 8, (sb, 256), jnp.float32, 0)
            m1 = pltpu.matmul_pop(s * 8, (sb, 256), jnp.float32, 1)
            g = (pf_s[pl.ds(rf, sb), :] + pb_s[pl.ds(rb, sb), :]
                 + jnp.concatenate([m0, m1], axis=1))
            tg = jnp.tanh(g)
            sif = tg[:, 0:2 * hid2] * 0.5 + 0.5
            ig = sif[:, 0:hid2]
            fg = sif[:, hid2:2 * hid2]
            gg = tg[:, 2 * hid2:3 * hid2]
            og = tg[:, 3 * hid2:4 * hid2] * 0.5 + 0.5
            c = fg * cs[s] + ig * gg
            hf32 = og * jnp.tanh(c)
            new_c.append(c)
            new_hf.append(hf32)
            new_h.append(hf32.astype(jnp.bfloat16))
            if accumulate:
                new_a.append(acs[s] + hf32)
        if not accumulate:
            hcat = jnp.concatenate(new_h, axis=0)
            outs[0][pl.ds(t, 1)] = hcat.reshape(1, bh, hid2)
            outs[1][pl.ds(tb, 1)] = hcat.reshape(1, bh, hid2)
            new_a = list(acs)
        return (tuple(new_h), tuple(new_c), tuple(new_a))

    init = (tuple(h_s[pl.ds(s * sb, sb), :] for s in range(ns)),
            tuple(c_s[pl.ds(s * sb, sb), :] for s in range(ns)),
            tuple(a_s[pl.ds(s * sb, sb), :] for s in range(ns)))
    fin = jax.lax.fori_loop(0, tc, step, init)
    h_s[...] = jnp.concatenate(fin[0], axis=0)
    c_s[...] = jnp.concatenate(fin[1], axis=0)
    a_s[...] = jnp.concatenate(fin[2], axis=0)

    if accumulate:
        @pl.when(k == nt - 1)
        def _():
            outs[0][...] = jnp.concatenate(fin[2], axis=0) * (1.0 / t_total)


def _lstm_layer(ins, wpf, wpb, wc, bf, bb, bh, tc, accumulate):
    T, B, _ = ins[0].shape
    g8 = wc.shape[1]
    hid2 = g8 // 4
    nt = T // tc
    nb = B // bh
    n_in = len(ins)

    in_specs = []
    args = []
    for xr in ins:
        d = xr.shape[-1]
        in_specs.append(pl.BlockSpec((tc, bh, d), lambda b, k: (k, b, 0)))
        args.append(xr)
    for xr in ins:
        d = xr.shape[-1]
        in_specs.append(
            pl.BlockSpec((tc, bh, d), lambda b, k: (nt - 1 - k, b, 0)))
        args.append(xr)
    for w in (wpf, wpb, wc, bf, bb):
        in_specs.append(pl.BlockSpec(w.shape, lambda b, k: (0,) * w.ndim))
        args.append(w)

    if accumulate:
        out_shape = (jax.ShapeDtypeStruct((B, hid2), jnp.float32),)
        out_specs = [pl.BlockSpec((bh, hid2), lambda b, k: (b, 0))]
    else:
        out_shape = (jax.ShapeDtypeStruct((T, B, hid2), jnp.bfloat16),) * 2
        out_specs = [
            pl.BlockSpec((tc, bh, hid2), lambda b, k: (k, b, 0)),
            pl.BlockSpec((tc, bh, hid2), lambda b, k: (nt - 1 - k, b, 0)),
        ]

    scratch = [
        pltpu.VMEM((tc * bh, g8), jnp.float32),
        pltpu.VMEM((tc * bh, g8), jnp.float32),
        pltpu.VMEM((bh, hid2), jnp.bfloat16),
        pltpu.VMEM((bh, hid2), jnp.float32),
        pltpu.VMEM((bh, hid2), jnp.float32),
    ]
    out = pl.pallas_call(
        functools.partial(_lstm_body, tc, bh, hid2, n_in, accumulate, T, nt),
        grid=(nb, nt),
        in_specs=in_specs,
        out_specs=out_specs,
        out_shape=out_shape,
        scratch_shapes=scratch,
        compiler_params=pltpu.CompilerParams(
            dimension_semantics=("parallel", "arbitrary"),
            vmem_limit_bytes=56 * 1024 * 1024),
    )(*args)
    return out


def _spread(w, slot, hid):
    z = jnp.zeros(w.shape[:-1] + (hid,), w.dtype)
    parts = []
    for q in range(4):
        blk = w[..., q * hid:(q + 1) * hid]
        if q != 2:
            blk = blk * 0.5
        parts.extend([blk, z] if slot == 0 else [z, blk])
    return jnp.concatenate(parts, axis=-1)


def kernel(x, binpoints, w_ih_0, w_hh_0, b_ih_0, b_hh_0,
           w_ih_1, w_hh_1, b_ih_1, b_hh_1,
           w_ih_2, w_hh_2, b_ih_2, b_hh_2):
    B, T, NB = x.shape
    hid = w_hh_0.shape[-1]
    bh = B // 2
    tcf = 200 if T % 200 == 0 else T
    tc = 100 if T % 100 == 0 else T

    h0 = _frontend(x, binpoints, bh, tcf)
    h0t = jnp.transpose(h0, (1, 0, 2))

    def wiT(w):
        return jnp.transpose(w, (0, 2, 1))

    wi0T = jnp.pad(wiT(w_ih_0), ((0, 0), (0, _FPAD - _NFILT), (0, 0)))
    wi1T, wi2T = wiT(w_ih_1), wiT(w_ih_2)
    wh0T, wh1T, wh2T = wiT(w_hh_0), wiT(w_hh_1), wiT(w_hh_2)

    def combine_wh(whT):
        wc = jnp.concatenate(
            [_spread(whT[0], 0, hid), _spread(whT[1], 1, hid)], axis=0)
        return jnp.pad(wc, ((0, 256 - wc.shape[0]), (0, 0))).astype(jnp.bfloat16)

    def biases(b_ih, b_hh):
        bs = b_ih + b_hh
        return (_spread(bs[0].reshape(1, -1), 0, hid),
                _spread(bs[1].reshape(1, -1), 1, hid))

    def zero_rows(w, keep_top):
        top, bot = w[:hid], w[hid:]
        if keep_top:
            return jnp.concatenate([top, jnp.zeros_like(bot)], axis=0)
        return jnp.concatenate([jnp.zeros_like(top), bot], axis=0)

    def stack256(w_top, w_bot):
        return jnp.concatenate([w_top, w_bot], axis=0).astype(jnp.bfloat16)

    bf0, bb0 = biases(b_ih_0, b_hh_0)
    z128 = jnp.zeros((128, 512), jnp.float32)
    f0, r0 = _lstm_layer(
        [h0t],
        stack256(_spread(wi0T[0], 0, hid), z128),
        stack256(_spread(wi0T[1], 1, hid), z128),
        combine_wh(wh0T), bf0, bb0, bh, tc, False)

    def mk_io_weights(wT):
        wpf = stack256(_spread(zero_rows(wT[0], True), 0, hid),
                       _spread(zero_rows(wT[0], False), 0, hid))
        wpb = stack256(_spread(zero_rows(wT[1], True), 1, hid),
                       _spread(zero_rows(wT[1], False), 1, hid))
        return wpf, wpb

    wfs1, wbs1 = mk_io_weights(wi1T)
    bf1, bb1 = biases(b_ih_1, b_hh_1)
    f1, r1 = _lstm_layer([f0, r0], wfs1, wbs1, combine_wh(wh1T),
                         bf1, bb1, bh, tc, False)

    wfs2, wbs2 = mk_io_weights(wi2T)
    bf2, bb2 = biases(b_ih_2, b_hh_2)
    (mean_out,) = _lstm_layer([f1, r1], wfs2, wbs2, combine_wh(wh2T),
                              bf2, bb2, bh, tc, True)
    return mean_out


# speedup vs baseline: 1.2695x; 1.0069x over previous
import functools

import jax
import jax.numpy as jnp
from jax.experimental import pallas as pl
from jax.experimental.pallas import tpu as pltpu

_NFILT = 40
_FPAD = 128


def _frontend_body(nfilt, b0_ref, b1_ref, b2_ref, x_ref, o_ref):
    nb = x_ref.shape[-1]
    fp = o_ref.shape[-1]
    b0, b1, b2 = b0_ref[...], b1_ref[...], b2_ref[...]
    f0, f1, f2 = jnp.floor(b0), jnp.floor(b1), jnp.floor(b2)
    i = jax.lax.broadcasted_iota(jnp.int32, (nb, fp), 0).astype(jnp.float32)
    j = jax.lax.broadcasted_iota(jnp.int32, (nb, fp), 1)
    rise_m = (i >= f0) & (i < f1)
    fall_m = (i >= f1) & (i < f2)
    d1 = b1 - b0
    d2 = b2 - b1
    rv = (i - b0) / jnp.where(d1 > 0, d1, 1.0) ** 2
    fv = (b2 - i) / jnp.where(d2 > 0, d2, 1.0) ** 2
    val = jnp.where(fall_m, fv, jnp.where(rise_m, rv, 0.0))
    val = jnp.where(j < nfilt - 1, val, 0.0)
    val = jnp.where(j == 0, jnp.where(i == 0.0, 1.0, 0.0), val)

    bh, tc, _ = x_ref.shape
    xb = x_ref[...].reshape(bh * tc, nb)
    filt = jnp.dot(xb, val, preferred_element_type=jnp.float32)
    h = jnp.log(filt + 1e-10).astype(jnp.bfloat16)
    o_ref[...] = h.reshape(bh, tc, fp)


def _frontend(x, binpoints, bh, tcf):
    B, T, NB = x.shape
    nt = T // tcf
    pad = _FPAD - _NFILT
    b0 = jnp.pad(binpoints[0:_NFILT], (0, pad)).reshape(1, _FPAD)
    b1 = jnp.pad(binpoints[1:_NFILT + 1], (0, pad)).reshape(1, _FPAD)
    b2 = jnp.pad(binpoints[2:_NFILT + 2], (0, pad)).reshape(1, _FPAD)
    return pl.pallas_call(
        functools.partial(_frontend_body, _NFILT),
        grid=(B // bh, nt),
        in_specs=[
            pl.BlockSpec((1, _FPAD), lambda b, k: (0, 0)),
            pl.BlockSpec((1, _FPAD), lambda b, k: (0, 0)),
            pl.BlockSpec((1, _FPAD), lambda b, k: (0, 0)),
            pl.BlockSpec((bh, tcf, NB), lambda b, k: (b, k, 0)),
        ],
        out_specs=pl.BlockSpec((bh, tcf, _FPAD), lambda b, k: (b, k, 0)),
        out_shape=jax.ShapeDtypeStruct((B, T, _FPAD), jnp.bfloat16),
        compiler_params=pltpu.CompilerParams(
            dimension_semantics=("parallel", "arbitrary")),
    )(b0, b1, b2, x)


def _lstm_body(tc, bh, hid2, n_in, accumulate, t_total, nt, *refs):
    g8 = 4 * hid2
    xf = refs[0:n_in]
    xb = refs[n_in:2 * n_in]
    wpf, wpb, wc, bf, bb = refs[2 * n_in:2 * n_in + 5]
    n_out = 1 if accumulate else 2
    outs = refs[2 * n_in + 5:2 * n_in + 5 + n_out]
    pf_s, pb_s, h_s, c_s, a_s = refs[2 * n_in + 5 + n_out:]
    k = pl.program_id(1)
    rows = tc * bh
    mtc = 1
    for cand in range(tc, 0, -1):
        if tc % cand == 0 and cand * bh <= 1024:
            mtc = cand
            break
    mchunk = mtc * bh

    @pl.when(k == 0)
    def _():
        h_s[...] = jnp.zeros_like(h_s)
        c_s[...] = jnp.zeros_like(c_s)
        a_s[...] = jnp.zeros_like(a_s)

    def compute_pre(xs, w_ref, b_ref, out_ref):
        w = w_ref[...]
        pltpu.matmul_push_rhs(w[:, 0:256], 1, 0)
        pltpu.matmul_push_rhs(w[:, 256:512], 1, 1)
        bv = b_ref[...]
        for mt in range(0, tc, mtc):
            parts = [xr[pl.ds(mt, mtc), :, :].reshape(mchunk, xr.shape[-1])
                     for xr in xs]
            if len(parts) == 1:
                parts.append(jnp.zeros(
                    (mchunk, 256 - parts[0].shape[-1]), jnp.bfloat16))
            lhs = jnp.concatenate(parts, axis=1)
            lsr = 1 if mt == 0 else None
            pltpu.matmul_acc_lhs(0, lhs, 0, load_staged_rhs=lsr)
            pltpu.matmul_acc_lhs(0, lhs, 1, load_staged_rhs=lsr)
            p0 = pltpu.matmul_pop(0, (mchunk, 256), jnp.float32, 0)
            p1 = pltpu.matmul_pop(0, (mchunk, 256), jnp.float32, 1)
            m = mt * bh
            out_ref[m:m + mchunk, 0:256] = p0 + bv[:, 0:256]
            out_ref[m:m + mchunk, 256:512] = p1 + bv[:, 256:512]

    compute_pre(xf, wpf, bf, pf_s)
    compute_pre(xb, wpb, bb, pb_s)

    wc_v = wc[...]
    ns = 2
    sb = bh // ns
    zpad = jnp.zeros((sb, 128), jnp.bfloat16)
    pltpu.matmul_push_rhs(wc_v[:, 0:256], 0, 0)
    pltpu.matmul_push_rhs(wc_v[:, 256:512], 0, 1)
    zlatch = jnp.zeros((16, 256), jnp.bfloat16)
    pltpu.matmul_acc_lhs(0, zlatch, 0, load_staged_rhs=0)
    pltpu.matmul_acc_lhs(0, zlatch, 1, load_staged_rhs=0)
    _d0 = pltpu.matmul_pop(0, (16, 256), jnp.float32, 0)
    _d1 = pltpu.matmul_pop(0, (16, 256), jnp.float32, 1)

    def step(t, carry):
        hs, cs, acs = carry
        tb = tc - 1 - t
        new_h, new_c, new_a, new_hf = [], [], [], []
        for s in range(ns):
            hp = jnp.concatenate([hs[s], zpad], axis=1)
            pltpu.matmul_acc_lhs(s * 8, hp, 0)
            pltpu.matmul_acc_lhs(s * 8, hp, 1)
        for s in range(ns):
            rf = pl.multiple_of(t * bh + s * sb, sb)
            rb = pl.multiple_of(tb * bh + s * sb, sb)
            m0 = pltpu.matmul_pop(s * 8, (sb, 256), jnp.float32, 0)
            m1 = pltpu.matmul_pop(s * 8, (sb, 256), jnp.float32, 1)
            g = (pf_s[pl.ds(rf, sb), :] + pb_s[pl.ds(rb, sb), :]
                 + jnp.concatenate([m0, m1], axis=1))
            tg = jnp.tanh(g)
            sif = tg[:, 0:2 * hid2] * 0.5 + 0.5
            ig = sif[:, 0:hid2]
            fg = sif[:, hid2:2 * hid2]
            gg = tg[:, 2 * hid2:3 * hid2]
            og = tg[:, 3 * hid2:4 * hid2] * 0.5 + 0.5
            c = fg * cs[s] + ig * gg
            hf32 = og * jnp.tanh(c)
            new_c.append(c)
            new_hf.append(hf32)
            new_h.append(hf32.astype(jnp.bfloat16))
            if accumulate:
                new_a.append(acs[s] + hf32)
        if not accumulate:
            hcat = jnp.concatenate(new_h, axis=0)
            outs[0][pl.ds(t, 1)] = hcat.reshape(1, bh, hid2)
            outs[1][pl.ds(tb, 1)] = hcat.reshape(1, bh, hid2)
            new_a = list(acs)
        return (tuple(new_h), tuple(new_c), tuple(new_a))

    init = (tuple(h_s[pl.ds(s * sb, sb), :] for s in range(ns)),
            tuple(c_s[pl.ds(s * sb, sb), :] for s in range(ns)),
            tuple(a_s[pl.ds(s * sb, sb), :] for s in range(ns)))
    fin = jax.lax.fori_loop(0, tc, step, init)
    h_s[...] = jnp.concatenate(fin[0], axis=0)
    c_s[...] = jnp.concatenate(fin[1], axis=0)
    a_s[...] = jnp.concatenate(fin[2], axis=0)

    if accumulate:
        @pl.when(k == nt - 1)
        def _():
            outs[0][...] = jnp.concatenate(fin[2], axis=0) * (1.0 / t_total)


def _lstm_layer(ins, wpf, wpb, wc, bf, bb, bh, tc, accumulate):
    T, B, _ = ins[0].shape
    g8 = wc.shape[1]
    hid2 = g8 // 4
    nt = T // tc
    nb = B // bh
    n_in = len(ins)

    in_specs = []
    args = []
    for xr in ins:
        d = xr.shape[-1]
        in_specs.append(pl.BlockSpec((tc, bh, d), lambda b, k: (k, b, 0)))
        args.append(xr)
    for xr in ins:
        d = xr.shape[-1]
        in_specs.append(
            pl.BlockSpec((tc, bh, d), lambda b, k: (nt - 1 - k, b, 0)))
        args.append(xr)
    for w in (wpf, wpb, wc, bf, bb):
        in_specs.append(pl.BlockSpec(w.shape, lambda b, k: (0,) * w.ndim))
        args.append(w)

    if accumulate:
        out_shape = (jax.ShapeDtypeStruct((B, hid2), jnp.float32),)
        out_specs = [pl.BlockSpec((bh, hid2), lambda b, k: (b, 0))]
    else:
        out_shape = (jax.ShapeDtypeStruct((T, B, hid2), jnp.bfloat16),) * 2
        out_specs = [
            pl.BlockSpec((tc, bh, hid2), lambda b, k: (k, b, 0)),
            pl.BlockSpec((tc, bh, hid2), lambda b, k: (nt - 1 - k, b, 0)),
        ]

    scratch = [
        pltpu.VMEM((tc * bh, g8), jnp.float32),
        pltpu.VMEM((tc * bh, g8), jnp.float32),
        pltpu.VMEM((bh, hid2), jnp.bfloat16),
        pltpu.VMEM((bh, hid2), jnp.float32),
        pltpu.VMEM((bh, hid2), jnp.float32),
    ]
    out = pl.pallas_call(
        functools.partial(_lstm_body, tc, bh, hid2, n_in, accumulate, T, nt),
        grid=(nb, nt),
        in_specs=in_specs,
        out_specs=out_specs,
        out_shape=out_shape,
        scratch_shapes=scratch,
        compiler_params=pltpu.CompilerParams(
            dimension_semantics=("parallel", "arbitrary"),
            vmem_limit_bytes=56 * 1024 * 1024),
    )(*args)
    return out


def _spread(w, slot, hid):
    z = jnp.zeros(w.shape[:-1] + (hid,), w.dtype)
    parts = []
    for q in range(4):
        blk = w[..., q * hid:(q + 1) * hid]
        if q != 2:
            blk = blk * 0.5
        parts.extend([blk, z] if slot == 0 else [z, blk])
    return jnp.concatenate(parts, axis=-1)


def kernel(x, binpoints, w_ih_0, w_hh_0, b_ih_0, b_hh_0,
           w_ih_1, w_hh_1, b_ih_1, b_hh_1,
           w_ih_2, w_hh_2, b_ih_2, b_hh_2):
    B, T, NB = x.shape
    hid = w_hh_0.shape[-1]
    bh = B // 2
    tcf = 200 if T % 200 == 0 else T
    tc = 250 if T % 250 == 0 else T

    h0 = _frontend(x, binpoints, bh, tcf)
    h0t = jnp.transpose(h0, (1, 0, 2))

    def wiT(w):
        return jnp.transpose(w, (0, 2, 1))

    wi0T = jnp.pad(wiT(w_ih_0), ((0, 0), (0, _FPAD - _NFILT), (0, 0)))
    wi1T, wi2T = wiT(w_ih_1), wiT(w_ih_2)
    wh0T, wh1T, wh2T = wiT(w_hh_0), wiT(w_hh_1), wiT(w_hh_2)

    def combine_wh(whT):
        wc = jnp.concatenate(
            [_spread(whT[0], 0, hid), _spread(whT[1], 1, hid)], axis=0)
        return jnp.pad(wc, ((0, 256 - wc.shape[0]), (0, 0))).astype(jnp.bfloat16)

    def biases(b_ih, b_hh):
        bs = b_ih + b_hh
        return (_spread(bs[0].reshape(1, -1), 0, hid),
                _spread(bs[1].reshape(1, -1), 1, hid))

    def zero_rows(w, keep_top):
        top, bot = w[:hid], w[hid:]
        if keep_top:
            return jnp.concatenate([top, jnp.zeros_like(bot)], axis=0)
        return jnp.concatenate([jnp.zeros_like(top), bot], axis=0)

    def stack256(w_top, w_bot):
        return jnp.concatenate([w_top, w_bot], axis=0).astype(jnp.bfloat16)

    bf0, bb0 = biases(b_ih_0, b_hh_0)
    z128 = jnp.zeros((128, 512), jnp.float32)
    f0, r0 = _lstm_layer(
        [h0t],
        stack256(_spread(wi0T[0], 0, hid), z128),
        stack256(_spread(wi0T[1], 1, hid), z128),
        combine_wh(wh0T), bf0, bb0, bh, tc, False)

    def mk_io_weights(wT):
        wpf = stack256(_spread(zero_rows(wT[0], True), 0, hid),
                       _spread(zero_rows(wT[0], False), 0, hid))
        wpb = stack256(_spread(zero_rows(wT[1], True), 1, hid),
                       _spread(zero_rows(wT[1], False), 1, hid))
        return wpf, wpb

    wfs1, wbs1 = mk_io_weights(wi1T)
    bf1, bb1 = biases(b_ih_1, b_hh_1)
    f1, r1 = _lstm_layer([f0, r0], wfs1, wbs1, combine_wh(wh1T),
                         bf1, bb1, bh, tc, False)

    wfs2, wbs2 = mk_io_weights(wi2T)
    bf2, bb2 = biases(b_ih_2, b_hh_2)
    (mean_out,) = _lstm_layer([f1, r1], wfs2, wbs2, combine_wh(wh2T),
                              bf2, bb2, bh, tc, True)
    return mean_out


# speedup vs baseline: 1.2890x; 1.0153x over previous
import functools

import jax
import jax.numpy as jnp
from jax.experimental import pallas as pl
from jax.experimental.pallas import tpu as pltpu

_NFILT = 40
_FPAD = 128


def _frontend_body(nfilt, b0_ref, b1_ref, b2_ref, x_ref, o_ref):
    nb = x_ref.shape[-1]
    fp = o_ref.shape[-1]
    b0, b1, b2 = b0_ref[...], b1_ref[...], b2_ref[...]
    f0, f1, f2 = jnp.floor(b0), jnp.floor(b1), jnp.floor(b2)
    i = jax.lax.broadcasted_iota(jnp.int32, (nb, fp), 0).astype(jnp.float32)
    j = jax.lax.broadcasted_iota(jnp.int32, (nb, fp), 1)
    rise_m = (i >= f0) & (i < f1)
    fall_m = (i >= f1) & (i < f2)
    d1 = b1 - b0
    d2 = b2 - b1
    rv = (i - b0) / jnp.where(d1 > 0, d1, 1.0) ** 2
    fv = (b2 - i) / jnp.where(d2 > 0, d2, 1.0) ** 2
    val = jnp.where(fall_m, fv, jnp.where(rise_m, rv, 0.0))
    val = jnp.where(j < nfilt - 1, val, 0.0)
    val = jnp.where(j == 0, jnp.where(i == 0.0, 1.0, 0.0), val)

    bh, tc, _ = x_ref.shape
    xb = x_ref[...].reshape(bh * tc, nb)
    filt = jnp.dot(xb, val, preferred_element_type=jnp.float32)
    h = jnp.log(filt + 1e-10).astype(jnp.bfloat16)
    o_ref[...] = h.reshape(bh, tc, fp)


def _frontend(x, binpoints, bh, tcf):
    B, T, NB = x.shape
    nt = T // tcf
    pad = _FPAD - _NFILT
    b0 = jnp.pad(binpoints[0:_NFILT], (0, pad)).reshape(1, _FPAD)
    b1 = jnp.pad(binpoints[1:_NFILT + 1], (0, pad)).reshape(1, _FPAD)
    b2 = jnp.pad(binpoints[2:_NFILT + 2], (0, pad)).reshape(1, _FPAD)
    return pl.pallas_call(
        functools.partial(_frontend_body, _NFILT),
        grid=(B // bh, nt),
        in_specs=[
            pl.BlockSpec((1, _FPAD), lambda b, k: (0, 0)),
            pl.BlockSpec((1, _FPAD), lambda b, k: (0, 0)),
            pl.BlockSpec((1, _FPAD), lambda b, k: (0, 0)),
            pl.BlockSpec((bh, tcf, NB), lambda b, k: (b, k, 0)),
        ],
        out_specs=pl.BlockSpec((bh, tcf, _FPAD), lambda b, k: (b, k, 0)),
        out_shape=jax.ShapeDtypeStruct((B, T, _FPAD), jnp.bfloat16),
        compiler_params=pltpu.CompilerParams(
            dimension_semantics=("parallel", "arbitrary")),
    )(b0, b1, b2, x)


def _lstm_body(tc, bh, hid2, n_in, accumulate, t_total, nt, *refs):
    g8 = 4 * hid2
    xf = refs[0:n_in]
    xb = refs[n_in:2 * n_in]
    wpf, wpb, wc, bf, bb = refs[2 * n_in:2 * n_in + 5]
    n_out = 1 if accumulate else 2
    outs = refs[2 * n_in + 5:2 * n_in + 5 + n_out]
    pf_s, pb_s, h_s, c_s, a_s = refs[2 * n_in + 5 + n_out:]
    k = pl.program_id(1)
    rows = tc * bh
    mtc = 1
    for cand in range(tc, 0, -1):
        if tc % cand == 0 and cand * bh <= 1024:
            mtc = cand
            break
    mchunk = mtc * bh

    @pl.when(k == 0)
    def _():
        h_s[...] = jnp.zeros_like(h_s)
        c_s[...] = jnp.zeros_like(c_s)
        a_s[...] = jnp.zeros_like(a_s)

    def compute_pre(xs, w_ref, b_ref, out_ref):
        w = w_ref[...]
        pltpu.matmul_push_rhs(w[:, 0:256], 1, 0)
        pltpu.matmul_push_rhs(w[:, 256:512], 1, 1)
        bv = b_ref[...]
        for mt in range(0, tc, mtc):
            parts = [xr[pl.ds(mt, mtc), :, :].reshape(mchunk, xr.shape[-1])
                     for xr in xs]
            if len(parts) == 1:
                parts.append(jnp.zeros(
                    (mchunk, 256 - parts[0].shape[-1]), jnp.bfloat16))
            lhs = jnp.concatenate(parts, axis=1)
            lsr = 1 if mt == 0 else None
            pltpu.matmul_acc_lhs(0, lhs, 0, load_staged_rhs=lsr)
            pltpu.matmul_acc_lhs(0, lhs, 1, load_staged_rhs=lsr)
            p0 = pltpu.matmul_pop(0, (mchunk, 256), jnp.float32, 0)
            p1 = pltpu.matmul_pop(0, (mchunk, 256), jnp.float32, 1)
            m = mt * bh
            out_ref[m:m + mchunk, 0:256] = p0 + bv[:, 0:256]
            out_ref[m:m + mchunk, 256:512] = p1 + bv[:, 256:512]

    compute_pre(xf, wpf, bf, pf_s)
    compute_pre(xb, wpb, bb, pb_s)

    wc_v = wc[...]
    ns = 2
    sb = bh // ns
    zpad = jnp.zeros((sb, 128), jnp.bfloat16)
    pltpu.matmul_push_rhs(wc_v[:, 0:256], 0, 0)
    pltpu.matmul_push_rhs(wc_v[:, 256:512], 0, 1)
    zlatch = jnp.zeros((16, 256), jnp.bfloat16)
    pltpu.matmul_acc_lhs(0, zlatch, 0, load_staged_rhs=0)
    pltpu.matmul_acc_lhs(0, zlatch, 1, load_staged_rhs=0)
    _d0 = pltpu.matmul_pop(0, (16, 256), jnp.float32, 0)
    _d1 = pltpu.matmul_pop(0, (16, 256), jnp.float32, 1)

    def substep(t, carry):
        hs, cs, acs = carry
        tb = tc - 1 - t
        new_h, new_c, new_a, new_hf = [], [], [], []
        for s in range(ns):
            hp = jnp.concatenate([hs[s], zpad], axis=1)
            pltpu.matmul_acc_lhs(s * 8, hp, 0)
            pltpu.matmul_acc_lhs(s * 8, hp, 1)
        for s in range(ns):
            rf = pl.multiple_of(t * bh + s * sb, sb)
            rb = pl.multiple_of(tb * bh + s * sb, sb)
            m0 = pltpu.matmul_pop(s * 8, (sb, 256), jnp.float32, 0)
            m1 = pltpu.matmul_pop(s * 8, (sb, 256), jnp.float32, 1)
            g = (pf_s[pl.ds(rf, sb), :] + pb_s[pl.ds(rb, sb), :]
                 + jnp.concatenate([m0, m1], axis=1))
            tg = jnp.tanh(g)
            sif = tg[:, 0:2 * hid2] * 0.5 + 0.5
            ig = sif[:, 0:hid2]
            fg = sif[:, hid2:2 * hid2]
            gg = tg[:, 2 * hid2:3 * hid2]
            og = tg[:, 3 * hid2:4 * hid2] * 0.5 + 0.5
            c = fg * cs[s] + ig * gg
            hf32 = og * jnp.tanh(c)
            new_c.append(c)
            new_hf.append(hf32)
            new_h.append(hf32.astype(jnp.bfloat16))
            if accumulate:
                new_a.append(acs[s] + hf32)
        if not accumulate:
            hcat = jnp.concatenate(new_h, axis=0)
            outs[0][pl.ds(t, 1)] = hcat.reshape(1, bh, hid2)
            outs[1][pl.ds(tb, 1)] = hcat.reshape(1, bh, hid2)
            new_a = list(acs)
        return (tuple(new_h), tuple(new_c), tuple(new_a))

    unroll = 2 if tc % 2 == 0 else 1

    def step(ti, carry):
        for u in range(unroll):
            carry = substep(ti * unroll + u, carry)
        return carry

    init = (tuple(h_s[pl.ds(s * sb, sb), :] for s in range(ns)),
            tuple(c_s[pl.ds(s * sb, sb), :] for s in range(ns)),
            tuple(a_s[pl.ds(s * sb, sb), :] for s in range(ns)))
    fin = jax.lax.fori_loop(0, tc // unroll, step, init)
    h_s[...] = jnp.concatenate(fin[0], axis=0)
    c_s[...] = jnp.concatenate(fin[1], axis=0)
    a_s[...] = jnp.concatenate(fin[2], axis=0)

    if accumulate:
        @pl.when(k == nt - 1)
        def _():
            outs[0][...] = jnp.concatenate(fin[2], axis=0) * (1.0 / t_total)


def _lstm_layer(ins, wpf, wpb, wc, bf, bb, bh, tc, accumulate):
    T, B, _ = ins[0].shape
    g8 = wc.shape[1]
    hid2 = g8 // 4
    nt = T // tc
    nb = B // bh
    n_in = len(ins)

    in_specs = []
    args = []
    for xr in ins:
        d = xr.shape[-1]
        in_specs.append(pl.BlockSpec((tc, bh, d), lambda b, k: (k, b, 0)))
        args.append(xr)
    for xr in ins:
        d = xr.shape[-1]
        in_specs.append(
            pl.BlockSpec((tc, bh, d), lambda b, k: (nt - 1 - k, b, 0)))
        args.append(xr)
    for w in (wpf, wpb, wc, bf, bb):
        in_specs.append(pl.BlockSpec(w.shape, lambda b, k: (0,) * w.ndim))
        args.append(w)

    if accumulate:
        out_shape = (jax.ShapeDtypeStruct((B, hid2), jnp.float32),)
        out_specs = [pl.BlockSpec((bh, hid2), lambda b, k: (b, 0))]
    else:
        out_shape = (jax.ShapeDtypeStruct((T, B, hid2), jnp.bfloat16),) * 2
        out_specs = [
            pl.BlockSpec((tc, bh, hid2), lambda b, k: (k, b, 0)),
            pl.BlockSpec((tc, bh, hid2), lambda b, k: (nt - 1 - k, b, 0)),
        ]

    scratch = [
        pltpu.VMEM((tc * bh, g8), jnp.float32),
        pltpu.VMEM((tc * bh, g8), jnp.float32),
        pltpu.VMEM((bh, hid2), jnp.bfloat16),
        pltpu.VMEM((bh, hid2), jnp.float32),
        pltpu.VMEM((bh, hid2), jnp.float32),
    ]
    out = pl.pallas_call(
        functools.partial(_lstm_body, tc, bh, hid2, n_in, accumulate, T, nt),
        grid=(nb, nt),
        in_specs=in_specs,
        out_specs=out_specs,
        out_shape=out_shape,
        scratch_shapes=scratch,
        compiler_params=pltpu.CompilerParams(
            dimension_semantics=("parallel", "arbitrary"),
            vmem_limit_bytes=56 * 1024 * 1024),
    )(*args)
    return out


def _spread(w, slot, hid):
    z = jnp.zeros(w.shape[:-1] + (hid,), w.dtype)
    parts = []
    for q in range(4):
        blk = w[..., q * hid:(q + 1) * hid]
        if q != 2:
            blk = blk * 0.5
        parts.extend([blk, z] if slot == 0 else [z, blk])
    return jnp.concatenate(parts, axis=-1)


def kernel(x, binpoints, w_ih_0, w_hh_0, b_ih_0, b_hh_0,
           w_ih_1, w_hh_1, b_ih_1, b_hh_1,
           w_ih_2, w_hh_2, b_ih_2, b_hh_2):
    B, T, NB = x.shape
    hid = w_hh_0.shape[-1]
    bh = B // 2
    tcf = 200 if T % 200 == 0 else T
    tc = 250 if T % 250 == 0 else T

    h0 = _frontend(x, binpoints, bh, tcf)
    h0t = jnp.transpose(h0, (1, 0, 2))

    def wiT(w):
        return jnp.transpose(w, (0, 2, 1))

    wi0T = jnp.pad(wiT(w_ih_0), ((0, 0), (0, _FPAD - _NFILT), (0, 0)))
    wi1T, wi2T = wiT(w_ih_1), wiT(w_ih_2)
    wh0T, wh1T, wh2T = wiT(w_hh_0), wiT(w_hh_1), wiT(w_hh_2)

    def combine_wh(whT):
        wc = jnp.concatenate(
            [_spread(whT[0], 0, hid), _spread(whT[1], 1, hid)], axis=0)
        return jnp.pad(wc, ((0, 256 - wc.shape[0]), (0, 0))).astype(jnp.bfloat16)

    def biases(b_ih, b_hh):
        bs = b_ih + b_hh
        return (_spread(bs[0].reshape(1, -1), 0, hid),
                _spread(bs[1].reshape(1, -1), 1, hid))

    def zero_rows(w, keep_top):
        top, bot = w[:hid], w[hid:]
        if keep_top:
            return jnp.concatenate([top, jnp.zeros_like(bot)], axis=0)
        return jnp.concatenate([jnp.zeros_like(top), bot], axis=0)

    def stack256(w_top, w_bot):
        return jnp.concatenate([w_top, w_bot], axis=0).astype(jnp.bfloat16)

    bf0, bb0 = biases(b_ih_0, b_hh_0)
    z128 = jnp.zeros((128, 512), jnp.float32)
    f0, r0 = _lstm_layer(
        [h0t],
        stack256(_spread(wi0T[0], 0, hid), z128),
        stack256(_spread(wi0T[1], 1, hid), z128),
        combine_wh(wh0T), bf0, bb0, bh, tc, False)

    def mk_io_weights(wT):
        wpf = stack256(_spread(zero_rows(wT[0], True), 0, hid),
                       _spread(zero_rows(wT[0], False), 0, hid))
        wpb = stack256(_spread(zero_rows(wT[1], True), 1, hid),
                       _spread(zero_rows(wT[1], False), 1, hid))
        return wpf, wpb

    wfs1, wbs1 = mk_io_weights(wi1T)
    bf1, bb1 = biases(b_ih_1, b_hh_1)
    f1, r1 = _lstm_layer([f0, r0], wfs1, wbs1, combine_wh(wh1T),
                         bf1, bb1, bh, tc, False)

    wfs2, wbs2 = mk_io_weights(wi2T)
    bf2, bb2 = biases(b_ih_2, b_hh_2)
    (mean_out,) = _lstm_layer([f1, r1], wfs2, wbs2, combine_wh(wh2T),
                              bf2, bb2, bh, tc, True)
    return mean_out


# speedup vs baseline: 1.3048x; 1.0123x over previous
import functools

import jax
import jax.numpy as jnp
from jax.experimental import pallas as pl
from jax.experimental.pallas import tpu as pltpu

_NFILT = 40
_FPAD = 128


def _frontend_body(nfilt, b0_ref, b1_ref, b2_ref, x_ref, o_ref):
    nb = x_ref.shape[-1]
    fp = o_ref.shape[-1]
    b0, b1, b2 = b0_ref[...], b1_ref[...], b2_ref[...]
    f0, f1, f2 = jnp.floor(b0), jnp.floor(b1), jnp.floor(b2)
    i = jax.lax.broadcasted_iota(jnp.int32, (nb, fp), 0).astype(jnp.float32)
    j = jax.lax.broadcasted_iota(jnp.int32, (nb, fp), 1)
    rise_m = (i >= f0) & (i < f1)
    fall_m = (i >= f1) & (i < f2)
    d1 = b1 - b0
    d2 = b2 - b1
    rv = (i - b0) / jnp.where(d1 > 0, d1, 1.0) ** 2
    fv = (b2 - i) / jnp.where(d2 > 0, d2, 1.0) ** 2
    val = jnp.where(fall_m, fv, jnp.where(rise_m, rv, 0.0))
    val = jnp.where(j < nfilt - 1, val, 0.0)
    val = jnp.where(j == 0, jnp.where(i == 0.0, 1.0, 0.0), val)

    bh, tc, _ = x_ref.shape
    xb = x_ref[...].reshape(bh * tc, nb)
    filt = jnp.dot(xb, val, preferred_element_type=jnp.float32)
    h = jnp.log(filt + 1e-10).astype(jnp.bfloat16)
    o_ref[...] = h.reshape(bh, tc, fp)


def _frontend(x, binpoints, bh, tcf):
    B, T, NB = x.shape
    nt = T // tcf
    pad = _FPAD - _NFILT
    b0 = jnp.pad(binpoints[0:_NFILT], (0, pad)).reshape(1, _FPAD)
    b1 = jnp.pad(binpoints[1:_NFILT + 1], (0, pad)).reshape(1, _FPAD)
    b2 = jnp.pad(binpoints[2:_NFILT + 2], (0, pad)).reshape(1, _FPAD)
    return pl.pallas_call(
        functools.partial(_frontend_body, _NFILT),
        grid=(B // bh, nt),
        in_specs=[
            pl.BlockSpec((1, _FPAD), lambda b, k: (0, 0)),
            pl.BlockSpec((1, _FPAD), lambda b, k: (0, 0)),
            pl.BlockSpec((1, _FPAD), lambda b, k: (0, 0)),
            pl.BlockSpec((bh, tcf, NB), lambda b, k: (b, k, 0)),
        ],
        out_specs=pl.BlockSpec((bh, tcf, _FPAD), lambda b, k: (b, k, 0)),
        out_shape=jax.ShapeDtypeStruct((B, T, _FPAD), jnp.bfloat16),
        compiler_params=pltpu.CompilerParams(
            dimension_semantics=("parallel", "arbitrary")),
    )(b0, b1, b2, x)


def _lstm_body(tc, bh, hid2, n_in, accumulate, t_total, nt, *refs):
    g8 = 4 * hid2
    xf = refs[0:n_in]
    xb = refs[n_in:2 * n_in]
    wpf, wpb, wc, bf, bb = refs[2 * n_in:2 * n_in + 5]
    n_out = 1 if accumulate else 2
    outs = refs[2 * n_in + 5:2 * n_in + 5 + n_out]
    pf_s, pb_s, h_s, c_s, a_s = refs[2 * n_in + 5 + n_out:]
    k = pl.program_id(1)
    rows = tc * bh
    mtc = 1
    for cand in range(tc, 0, -1):
        if tc % cand == 0 and cand * bh <= 1024:
            mtc = cand
            break
    mchunk = mtc * bh

    @pl.when(k == 0)
    def _():
        h_s[...] = jnp.zeros_like(h_s)
        c_s[...] = jnp.zeros_like(c_s)
        a_s[...] = jnp.zeros_like(a_s)

    def compute_pre(xs, w_ref, b_ref, out_ref):
        w = w_ref[...]
        pltpu.matmul_push_rhs(w[:, 0:256], 1, 0)
        pltpu.matmul_push_rhs(w[:, 256:512], 1, 1)
        bv = b_ref[...]
        for mt in range(0, tc, mtc):
            parts = [xr[pl.ds(mt, mtc), :, :].reshape(mchunk, xr.shape[-1])
                     for xr in xs]
            if len(parts) == 1:
                parts.append(jnp.zeros(
                    (mchunk, 256 - parts[0].shape[-1]), jnp.bfloat16))
            lhs = jnp.concatenate(parts, axis=1)
            lsr = 1 if mt == 0 else None
            pltpu.matmul_acc_lhs(0, lhs, 0, load_staged_rhs=lsr)
            pltpu.matmul_acc_lhs(0, lhs, 1, load_staged_rhs=lsr)
            p0 = pltpu.matmul_pop(0, (mchunk, 256), jnp.float32, 0)
            p1 = pltpu.matmul_pop(0, (mchunk, 256), jnp.float32, 1)
            m = mt * bh
            out_ref[m:m + mchunk, 0:256] = p0 + bv[:, 0:256]
            out_ref[m:m + mchunk, 256:512] = p1 + bv[:, 256:512]

    compute_pre(xf, wpf, bf, pf_s)
    compute_pre(xb, wpb, bb, pb_s)

    wc_v = wc[...]
    ns = 2
    sb = bh // ns
    zpad = jnp.zeros((sb, 128), jnp.bfloat16)
    pltpu.matmul_push_rhs(wc_v[:, 0:256], 0, 0)
    pltpu.matmul_push_rhs(wc_v[:, 256:512], 0, 1)
    zlatch = jnp.zeros((16, 256), jnp.bfloat16)
    pltpu.matmul_acc_lhs(0, zlatch, 0, load_staged_rhs=0)
    pltpu.matmul_acc_lhs(0, zlatch, 1, load_staged_rhs=0)
    _d0 = pltpu.matmul_pop(0, (16, 256), jnp.float32, 0)
    _d1 = pltpu.matmul_pop(0, (16, 256), jnp.float32, 1)

    def issue_acc(s, h):
        hp = jnp.concatenate([h, zpad], axis=1)
        pltpu.matmul_acc_lhs(s * 8, hp, 0)
        pltpu.matmul_acc_lhs(s * 8, hp, 1)

    def substep(t, carry):
        hs, cs, acs = carry
        tb = tc - 1 - t
        new_h, new_c, new_a, new_hf = [], [], [], []
        for s in range(ns):
            rf = pl.multiple_of(t * bh + s * sb, sb)
            rb = pl.multiple_of(tb * bh + s * sb, sb)
            m0 = pltpu.matmul_pop(s * 8, (sb, 256), jnp.float32, 0)
            m1 = pltpu.matmul_pop(s * 8, (sb, 256), jnp.float32, 1)
            g = (pf_s[pl.ds(rf, sb), :] + pb_s[pl.ds(rb, sb), :]
                 + jnp.concatenate([m0, m1], axis=1))
            tg = jnp.tanh(g)
            sif = tg[:, 0:2 * hid2] * 0.5 + 0.5
            ig = sif[:, 0:hid2]
            fg = sif[:, hid2:2 * hid2]
            gg = tg[:, 2 * hid2:3 * hid2]
            og = tg[:, 3 * hid2:4 * hid2] * 0.5 + 0.5
            c = fg * cs[s] + ig * gg
            hf32 = og * jnp.tanh(c)
            hbf = hf32.astype(jnp.bfloat16)
            issue_acc(s, hbf)
            new_c.append(c)
            new_hf.append(hf32)
            new_h.append(hbf)
            if accumulate:
                new_a.append(acs[s] + hf32)
        if not accumulate:
            hcat = jnp.concatenate(new_h, axis=0)
            outs[0][pl.ds(t, 1)] = hcat.reshape(1, bh, hid2)
            outs[1][pl.ds(tb, 1)] = hcat.reshape(1, bh, hid2)
            new_a = list(acs)
        return (tuple(new_h), tuple(new_c), tuple(new_a))

    unroll = 2 if tc % 2 == 0 else 1

    def step(ti, carry):
        for u in range(unroll):
            carry = substep(ti * unroll + u, carry)
        return carry

    init = (tuple(h_s[pl.ds(s * sb, sb), :] for s in range(ns)),
            tuple(c_s[pl.ds(s * sb, sb), :] for s in range(ns)),
            tuple(a_s[pl.ds(s * sb, sb), :] for s in range(ns)))
    for s in range(ns):
        issue_acc(s, init[0][s])
    fin = jax.lax.fori_loop(0, tc // unroll, step, init)
    for s in range(ns):
        _x0 = pltpu.matmul_pop(s * 8, (sb, 256), jnp.float32, 0)
        _x1 = pltpu.matmul_pop(s * 8, (sb, 256), jnp.float32, 1)
    h_s[...] = jnp.concatenate(fin[0], axis=0)
    c_s[...] = jnp.concatenate(fin[1], axis=0)
    a_s[...] = jnp.concatenate(fin[2], axis=0)

    if accumulate:
        @pl.when(k == nt - 1)
        def _():
            outs[0][...] = jnp.concatenate(fin[2], axis=0) * (1.0 / t_total)


def _lstm_layer(ins, wpf, wpb, wc, bf, bb, bh, tc, accumulate):
    T, B, _ = ins[0].shape
    g8 = wc.shape[1]
    hid2 = g8 // 4
    nt = T // tc
    nb = B // bh
    n_in = len(ins)

    in_specs = []
    args = []
    for xr in ins:
        d = xr.shape[-1]
        in_specs.append(pl.BlockSpec((tc, bh, d), lambda b, k: (k, b, 0)))
        args.append(xr)
    for xr in ins:
        d = xr.shape[-1]
        in_specs.append(
            pl.BlockSpec((tc, bh, d), lambda b, k: (nt - 1 - k, b, 0)))
        args.append(xr)
    for w in (wpf, wpb, wc, bf, bb):
        in_specs.append(pl.BlockSpec(w.shape, lambda b, k: (0,) * w.ndim))
        args.append(w)

    if accumulate:
        out_shape = (jax.ShapeDtypeStruct((B, hid2), jnp.float32),)
        out_specs = [pl.BlockSpec((bh, hid2), lambda b, k: (b, 0))]
    else:
        out_shape = (jax.ShapeDtypeStruct((T, B, hid2), jnp.bfloat16),) * 2
        out_specs = [
            pl.BlockSpec((tc, bh, hid2), lambda b, k: (k, b, 0)),
            pl.BlockSpec((tc, bh, hid2), lambda b, k: (nt - 1 - k, b, 0)),
        ]

    scratch = [
        pltpu.VMEM((tc * bh, g8), jnp.float32),
        pltpu.VMEM((tc * bh, g8), jnp.float32),
        pltpu.VMEM((bh, hid2), jnp.bfloat16),
        pltpu.VMEM((bh, hid2), jnp.float32),
        pltpu.VMEM((bh, hid2), jnp.float32),
    ]
    out = pl.pallas_call(
        functools.partial(_lstm_body, tc, bh, hid2, n_in, accumulate, T, nt),
        grid=(nb, nt),
        in_specs=in_specs,
        out_specs=out_specs,
        out_shape=out_shape,
        scratch_shapes=scratch,
        compiler_params=pltpu.CompilerParams(
            dimension_semantics=("parallel", "arbitrary"),
            vmem_limit_bytes=56 * 1024 * 1024),
    )(*args)
    return out


def _spread(w, slot, hid):
    z = jnp.zeros(w.shape[:-1] + (hid,), w.dtype)
    parts = []
    for q in range(4):
        blk = w[..., q * hid:(q + 1) * hid]
        if q != 2:
            blk = blk * 0.5
        parts.extend([blk, z] if slot == 0 else [z, blk])
    return jnp.concatenate(parts, axis=-1)


def kernel(x, binpoints, w_ih_0, w_hh_0, b_ih_0, b_hh_0,
           w_ih_1, w_hh_1, b_ih_1, b_hh_1,
           w_ih_2, w_hh_2, b_ih_2, b_hh_2):
    B, T, NB = x.shape
    hid = w_hh_0.shape[-1]
    bh = B // 2
    tcf = 200 if T % 200 == 0 else T
    tc = 250 if T % 250 == 0 else T

    h0 = _frontend(x, binpoints, bh, tcf)
    h0t = jnp.transpose(h0, (1, 0, 2))

    def wiT(w):
        return jnp.transpose(w, (0, 2, 1))

    wi0T = jnp.pad(wiT(w_ih_0), ((0, 0), (0, _FPAD - _NFILT), (0, 0)))
    wi1T, wi2T = wiT(w_ih_1), wiT(w_ih_2)
    wh0T, wh1T, wh2T = wiT(w_hh_0), wiT(w_hh_1), wiT(w_hh_2)

    def combine_wh(whT):
        wc = jnp.concatenate(
            [_spread(whT[0], 0, hid), _spread(whT[1], 1, hid)], axis=0)
        return jnp.pad(wc, ((0, 256 - wc.shape[0]), (0, 0))).astype(jnp.bfloat16)

    def biases(b_ih, b_hh):
        bs = b_ih + b_hh
        return (_spread(bs[0].reshape(1, -1), 0, hid),
                _spread(bs[1].reshape(1, -1), 1, hid))

    def zero_rows(w, keep_top):
        top, bot = w[:hid], w[hid:]
        if keep_top:
            return jnp.concatenate([top, jnp.zeros_like(bot)], axis=0)
        return jnp.concatenate([jnp.zeros_like(top), bot], axis=0)

    def stack256(w_top, w_bot):
        return jnp.concatenate([w_top, w_bot], axis=0).astype(jnp.bfloat16)

    bf0, bb0 = biases(b_ih_0, b_hh_0)
    z128 = jnp.zeros((128, 512), jnp.float32)
    f0, r0 = _lstm_layer(
        [h0t],
        stack256(_spread(wi0T[0], 0, hid), z128),
        stack256(_spread(wi0T[1], 1, hid), z128),
        combine_wh(wh0T), bf0, bb0, bh, tc, False)

    def mk_io_weights(wT):
        wpf = stack256(_spread(zero_rows(wT[0], True), 0, hid),
                       _spread(zero_rows(wT[0], False), 0, hid))
        wpb = stack256(_spread(zero_rows(wT[1], True), 1, hid),
                       _spread(zero_rows(wT[1], False), 1, hid))
        return wpf, wpb

    wfs1, wbs1 = mk_io_weights(wi1T)
    bf1, bb1 = biases(b_ih_1, b_hh_1)
    f1, r1 = _lstm_layer([f0, r0], wfs1, wbs1, combine_wh(wh1T),
                         bf1, bb1, bh, tc, False)

    wfs2, wbs2 = mk_io_weights(wi2T)
    bf2, bb2 = biases(b_ih_2, b_hh_2)
    (mean_out,) = _lstm_layer([f1, r1], wfs2, wbs2, combine_wh(wh2T),
                              bf2, bb2, bh, tc, True)
    return mean_out


# speedup vs baseline: 1.3056x; 1.0005x over previous
import functools

import jax
import jax.numpy as jnp
from jax.experimental import pallas as pl
from jax.experimental.pallas import tpu as pltpu

_NFILT = 40
_FPAD = 128


def _frontend_body(nfilt, b0_ref, b1_ref, b2_ref, x_ref, o_ref):
    nb = x_ref.shape[-1]
    fp = o_ref.shape[-1]
    b0, b1, b2 = b0_ref[...], b1_ref[...], b2_ref[...]
    f0, f1, f2 = jnp.floor(b0), jnp.floor(b1), jnp.floor(b2)
    i = jax.lax.broadcasted_iota(jnp.int32, (nb, fp), 0).astype(jnp.float32)
    j = jax.lax.broadcasted_iota(jnp.int32, (nb, fp), 1)
    rise_m = (i >= f0) & (i < f1)
    fall_m = (i >= f1) & (i < f2)
    d1 = b1 - b0
    d2 = b2 - b1
    rv = (i - b0) / jnp.where(d1 > 0, d1, 1.0) ** 2
    fv = (b2 - i) / jnp.where(d2 > 0, d2, 1.0) ** 2
    val = jnp.where(fall_m, fv, jnp.where(rise_m, rv, 0.0))
    val = jnp.where(j < nfilt - 1, val, 0.0)
    val = jnp.where(j == 0, jnp.where(i == 0.0, 1.0, 0.0), val)

    bh, tc, _ = x_ref.shape
    xb = x_ref[...].reshape(bh * tc, nb)
    filt = jnp.dot(xb, val, preferred_element_type=jnp.float32)
    h = jnp.log(filt + 1e-10).astype(jnp.bfloat16)
    o_ref[...] = h.reshape(bh, tc, fp)


def _frontend(x, binpoints, bh, tcf):
    B, T, NB = x.shape
    nt = T // tcf
    pad = _FPAD - _NFILT
    b0 = jnp.pad(binpoints[0:_NFILT], (0, pad)).reshape(1, _FPAD)
    b1 = jnp.pad(binpoints[1:_NFILT + 1], (0, pad)).reshape(1, _FPAD)
    b2 = jnp.pad(binpoints[2:_NFILT + 2], (0, pad)).reshape(1, _FPAD)
    return pl.pallas_call(
        functools.partial(_frontend_body, _NFILT),
        grid=(B // bh, nt),
        in_specs=[
            pl.BlockSpec((1, _FPAD), lambda b, k: (0, 0)),
            pl.BlockSpec((1, _FPAD), lambda b, k: (0, 0)),
            pl.BlockSpec((1, _FPAD), lambda b, k: (0, 0)),
            pl.BlockSpec((bh, tcf, NB), lambda b, k: (b, k, 0)),
        ],
        out_specs=pl.BlockSpec((bh, tcf, _FPAD), lambda b, k: (b, k, 0)),
        out_shape=jax.ShapeDtypeStruct((B, T, _FPAD), jnp.bfloat16),
        compiler_params=pltpu.CompilerParams(
            dimension_semantics=("parallel", "arbitrary")),
    )(b0, b1, b2, x)


def _lstm_body(tc, bh, hid2, n_in, accumulate, t_total, nt, *refs):
    g8 = 4 * hid2
    xf = refs[0:n_in]
    xb = refs[n_in:2 * n_in]
    wpf, wpb, wc, bf, bb = refs[2 * n_in:2 * n_in + 5]
    n_out = 1 if accumulate else 2
    outs = refs[2 * n_in + 5:2 * n_in + 5 + n_out]
    pf_s, pb_s, h_s, c_s, a_s = refs[2 * n_in + 5 + n_out:]
    k = pl.program_id(1)
    rows = tc * bh
    mtc = 1
    for cand in range(tc, 0, -1):
        if tc % cand == 0 and cand * bh <= 1024:
            mtc = cand
            break
    mchunk = mtc * bh

    @pl.when(k == 0)
    def _():
        h_s[...] = jnp.zeros_like(h_s)
        c_s[...] = jnp.zeros_like(c_s)
        a_s[...] = jnp.zeros_like(a_s)

    def compute_pre(xs, w_ref, b_ref, out_ref):
        w = w_ref[...]
        pltpu.matmul_push_rhs(w[:, 0:256], 1, 0)
        pltpu.matmul_push_rhs(w[:, 256:512], 1, 1)
        bv = b_ref[...]
        for mt in range(0, tc, mtc):
            parts = [xr[pl.ds(mt, mtc), :, :].reshape(mchunk, xr.shape[-1])
                     for xr in xs]
            if len(parts) == 1:
                parts.append(jnp.zeros(
                    (mchunk, 256 - parts[0].shape[-1]), jnp.bfloat16))
            lhs = jnp.concatenate(parts, axis=1)
            lsr = 1 if mt == 0 else None
            pltpu.matmul_acc_lhs(0, lhs, 0, load_staged_rhs=lsr)
            pltpu.matmul_acc_lhs(0, lhs, 1, load_staged_rhs=lsr)
            p0 = pltpu.matmul_pop(0, (mchunk, 256), jnp.float32, 0)
            p1 = pltpu.matmul_pop(0, (mchunk, 256), jnp.float32, 1)
            m = mt * bh
            out_ref[m:m + mchunk, 0:256] = p0 + bv[:, 0:256]
            out_ref[m:m + mchunk, 256:512] = p1 + bv[:, 256:512]

    compute_pre(xf, wpf, bf, pf_s)
    compute_pre(xb, wpb, bb, pb_s)

    wc_v = wc[...]
    ns = 2
    sb = bh // ns
    zpad = jnp.zeros((sb, 128), jnp.bfloat16)
    pltpu.matmul_push_rhs(wc_v[:, 0:256], 0, 0)
    pltpu.matmul_push_rhs(wc_v[:, 256:512], 0, 1)
    zlatch = jnp.zeros((16, 256), jnp.bfloat16)
    pltpu.matmul_acc_lhs(0, zlatch, 0, load_staged_rhs=0)
    pltpu.matmul_acc_lhs(0, zlatch, 1, load_staged_rhs=0)
    _d0 = pltpu.matmul_pop(0, (16, 256), jnp.float32, 0)
    _d1 = pltpu.matmul_pop(0, (16, 256), jnp.float32, 1)

    def issue_acc(s, h):
        hp = jnp.concatenate([h, zpad], axis=1)
        pltpu.matmul_acc_lhs(s * 8, hp, 0)
        pltpu.matmul_acc_lhs(s * 8, hp, 1)

    def substep(t, carry):
        hs, cs, acs = carry
        tb = tc - 1 - t
        new_h, new_c, new_a = [], [], []
        for s in range(ns):
            rf = pl.multiple_of(t * bh + s * sb, sb)
            rb = pl.multiple_of(tb * bh + s * sb, sb)
            m0 = pltpu.matmul_pop(s * 8, (sb, 256), jnp.float32, 0)
            m1 = pltpu.matmul_pop(s * 8, (sb, 256), jnp.float32, 1)
            g = (pf_s[pl.ds(rf, sb), :] + pb_s[pl.ds(rb, sb), :]
                 + jnp.concatenate([m0, m1], axis=1))
            tg = jnp.tanh(g)
            sif = tg[:, 0:2 * hid2] * 0.5 + 0.5
            ig = sif[:, 0:hid2]
            fg = sif[:, hid2:2 * hid2]
            gg = tg[:, 2 * hid2:3 * hid2]
            og = tg[:, 3 * hid2:4 * hid2] * 0.5 + 0.5
            c = fg * cs[s] + ig * gg
            hf32 = og * jnp.tanh(c)
            hbf = hf32.astype(jnp.bfloat16)
            issue_acc(s, hbf)
            new_c.append(c)
            new_h.append(hbf)
            if accumulate:
                new_a.append(acs[s] + hf32)
        if not accumulate:
            hcat = jnp.concatenate(new_h, axis=0)
            outs[0][pl.ds(t, 1)] = hcat.reshape(1, bh, hid2)
            outs[1][pl.ds(tb, 1)] = hcat.reshape(1, bh, hid2)
            new_a = list(acs)
        return (tuple(new_h), tuple(new_c), tuple(new_a))

    unroll = 2 if tc % 2 == 0 else 1

    def step(ti, carry):
        for u in range(unroll):
            carry = substep(ti * unroll + u, carry)
        return carry

    init = (tuple(h_s[pl.ds(s * sb, sb), :] for s in range(ns)),
            tuple(c_s[pl.ds(s * sb, sb), :] for s in range(ns)),
            tuple(a_s[pl.ds(s * sb, sb), :] for s in range(ns)))
    for s in range(ns):
        issue_acc(s, init[0][s])
    fin = jax.lax.fori_loop(0, tc // unroll, step, init)
    for s in range(ns):
        _x0 = pltpu.matmul_pop(s * 8, (sb, 256), jnp.float32, 0)
        _x1 = pltpu.matmul_pop(s * 8, (sb, 256), jnp.float32, 1)
    h_s[...] = jnp.concatenate(fin[0], axis=0)
    c_s[...] = jnp.concatenate(fin[1], axis=0)
    a_s[...] = jnp.concatenate(fin[2], axis=0)

    if accumulate:
        @pl.when(k == nt - 1)
        def _():
            outs[0][...] = jnp.concatenate(fin[2], axis=0) * (1.0 / t_total)


def _lstm_layer(ins, wpf, wpb, wc, bf, bb, bh, tc, accumulate):
    T, B, _ = ins[0].shape
    g8 = wc.shape[1]
    hid2 = g8 // 4
    nt = T // tc
    nb = B // bh
    n_in = len(ins)

    in_specs = []
    args = []
    for xr in ins:
        d = xr.shape[-1]
        in_specs.append(pl.BlockSpec((tc, bh, d), lambda b, k: (k, b, 0)))
        args.append(xr)
    for xr in ins:
        d = xr.shape[-1]
        in_specs.append(
            pl.BlockSpec((tc, bh, d), lambda b, k: (nt - 1 - k, b, 0)))
        args.append(xr)
    for w in (wpf, wpb, wc, bf, bb):
        in_specs.append(pl.BlockSpec(w.shape, lambda b, k: (0,) * w.ndim))
        args.append(w)

    if accumulate:
        out_shape = (jax.ShapeDtypeStruct((B, hid2), jnp.float32),)
        out_specs = [pl.BlockSpec((bh, hid2), lambda b, k: (b, 0))]
    else:
        out_shape = (jax.ShapeDtypeStruct((T, B, hid2), jnp.bfloat16),) * 2
        out_specs = [
            pl.BlockSpec((tc, bh, hid2), lambda b, k: (k, b, 0)),
            pl.BlockSpec((tc, bh, hid2), lambda b, k: (nt - 1 - k, b, 0)),
        ]

    scratch = [
        pltpu.VMEM((tc * bh, g8), jnp.float32),
        pltpu.VMEM((tc * bh, g8), jnp.float32),
        pltpu.VMEM((bh, hid2), jnp.bfloat16),
        pltpu.VMEM((bh, hid2), jnp.float32),
        pltpu.VMEM((bh, hid2), jnp.float32),
    ]
    out = pl.pallas_call(
        functools.partial(_lstm_body, tc, bh, hid2, n_in, accumulate, T, nt),
        grid=(nb, nt),
        in_specs=in_specs,
        out_specs=out_specs,
        out_shape=out_shape,
        scratch_shapes=scratch,
        compiler_params=pltpu.CompilerParams(
            dimension_semantics=("parallel", "arbitrary"),
            vmem_limit_bytes=56 * 1024 * 1024),
    )(*args)
    return out


def _spread(w, slot, hid):
    z = jnp.zeros(w.shape[:-1] + (hid,), w.dtype)
    parts = []
    for q in range(4):
        blk = w[..., q * hid:(q + 1) * hid]
        if q != 2:
            blk = blk * 0.5
        parts.extend([blk, z] if slot == 0 else [z, blk])
    return jnp.concatenate(parts, axis=-1)


def kernel(x, binpoints, w_ih_0, w_hh_0, b_ih_0, b_hh_0,
           w_ih_1, w_hh_1, b_ih_1, b_hh_1,
           w_ih_2, w_hh_2, b_ih_2, b_hh_2):
    B, T, NB = x.shape
    hid = w_hh_0.shape[-1]
    bh = B // 2
    tcf = 200 if T % 200 == 0 else T
    tc = 250 if T % 250 == 0 else T

    h0 = _frontend(x, binpoints, bh, tcf)
    h0t = jnp.transpose(h0, (1, 0, 2))

    def wiT(w):
        return jnp.transpose(w, (0, 2, 1))

    wi0T = jnp.pad(wiT(w_ih_0), ((0, 0), (0, _FPAD - _NFILT), (0, 0)))
    wi1T, wi2T = wiT(w_ih_1), wiT(w_ih_2)
    wh0T, wh1T, wh2T = wiT(w_hh_0), wiT(w_hh_1), wiT(w_hh_2)

    def combine_wh(whT):
        wc = jnp.concatenate(
            [_spread(whT[0], 0, hid), _spread(whT[1], 1, hid)], axis=0)
        return jnp.pad(wc, ((0, 256 - wc.shape[0]), (0, 0))).astype(jnp.bfloat16)

    def biases(b_ih, b_hh):
        bs = b_ih + b_hh
        return (_spread(bs[0].reshape(1, -1), 0, hid),
                _spread(bs[1].reshape(1, -1), 1, hid))

    def zero_rows(w, keep_top):
        top, bot = w[:hid], w[hid:]
        if keep_top:
            return jnp.concatenate([top, jnp.zeros_like(bot)], axis=0)
        return jnp.concatenate([jnp.zeros_like(top), bot], axis=0)

    def stack256(w_top, w_bot):
        return jnp.concatenate([w_top, w_bot], axis=0).astype(jnp.bfloat16)

    bf0, bb0 = biases(b_ih_0, b_hh_0)
    z128 = jnp.zeros((128, 512), jnp.float32)
    f0, r0 = _lstm_layer(
        [h0t],
        stack256(_spread(wi0T[0], 0, hid), z128),
        stack256(_spread(wi0T[1], 1, hid), z128),
        combine_wh(wh0T), bf0, bb0, bh, tc, False)

    def mk_io_weights(wT):
        wpf = stack256(_spread(zero_rows(wT[0], True), 0, hid),
                       _spread(zero_rows(wT[0], False), 0, hid))
        wpb = stack256(_spread(zero_rows(wT[1], True), 1, hid),
                       _spread(zero_rows(wT[1], False), 1, hid))
        return wpf, wpb

    wfs1, wbs1 = mk_io_weights(wi1T)
    bf1, bb1 = biases(b_ih_1, b_hh_1)
    f1, r1 = _lstm_layer([f0, r0], wfs1, wbs1, combine_wh(wh1T),
                         bf1, bb1, bh, tc, False)

    wfs2, wbs2 = mk_io_weights(wi2T)
    bf2, bb2 = biases(b_ih_2, b_hh_2)
    (mean_out,) = _lstm_layer([f1, r1], wfs2, wbs2, combine_wh(wh2T),
                              bf2, bb2, bh, tc, True)
    return mean_out
